# Initial kernel scaffold; baseline (speedup 1.0000x reference)
import jax
import jax.numpy as jnp
from jax.experimental import pallas as pl


def kernel(vectors, node_feats, node_specie, senders, receivers, W_sc, W_up, W_tp, W1, W2, W3, W_down):
    raise NotImplementedError("write your pallas kernel here")


# speedup vs baseline: 2.3348x; 2.3348x over previous
import functools

import numpy as np
import jax
import jax.numpy as jnp
from jax import lax
from jax.experimental import pallas as pl
from jax.experimental.pallas import tpu as pltpu
from jax.experimental.pallas import tpu_sc as plsc

N_NODES = 10000
N_EDGES = 160000
NF = 32
IR = 9
D = NF * IR
SH_DIM = 15
MSG_DIM = 2 * D
GATE_DIM = D + 2 * NF
CUTOFF = 3.0

_SIGMA = np.arange(D).reshape(NF, IR).T.reshape(-1)
_PI_FULL = np.concatenate([_SIGMA, D + _SIGMA])

_P_TILE = np.tile(np.eye(NF, dtype=np.float32), (1, IR))
_IRP = 384

_R1 = np.zeros((NF, 3 * NF), np.float32)
_R1[np.repeat(np.arange(NF), 3), np.arange(3 * NF)] = 1.0
_R2 = np.zeros((NF, 5 * NF), np.float32)
_R2[np.repeat(np.arange(NF), 5), np.arange(5 * NF)] = 1.0

_NC, _NS = 2, 16
_NW = _NC * _NS
_GPW = N_EDGES // _NW
_GCH = 128
_GFULL = _GPW // _GCH
_GTAIL = _GPW - _GFULL * _GCH
_SPS = N_EDGES // _NS
_SCH = 80
_SNCH = _SPS // _SCH
_GRP = MSG_DIM // 6
_GPC = 3
_RPS = N_NODES // _NS
_ZCH = 125


def _swish(x):
    return x / (1.0 + jnp.exp(-x))


def _node_pre_body(x_ref, spec_ref, wup_ref, wsc_ref, h_ref, sc_ref):
    x = x_ref[...]
    h_ref[...] = jnp.dot(x, wup_ref[...], preferred_element_type=jnp.float32)
    spec = spec_ref[...]
    acc = (spec == 0).astype(jnp.float32) * jnp.dot(
        x, wsc_ref[0], preferred_element_type=jnp.float32)
    for s in range(1, 5):
        acc = acc + (spec == s).astype(jnp.float32) * jnp.dot(
            x, wsc_ref[s], preferred_element_type=jnp.float32)
    sc_ref[...] = acc


def _node_pre(x, spec2d, wup, wsc):
    bn = 2000
    return pl.pallas_call(
        _node_pre_body,
        grid=(N_NODES // bn,),
        in_specs=[
            pl.BlockSpec((bn, D), lambda i: (i, 0)),
            pl.BlockSpec((bn, 1), lambda i: (i, 0)),
            pl.BlockSpec((D, D), lambda i: (0, 0)),
            pl.BlockSpec((5, D, D), lambda i: (0, 0, 0)),
        ],
        out_specs=[
            pl.BlockSpec((bn, D), lambda i: (i, 0)),
            pl.BlockSpec((bn, D), lambda i: (i, 0)),
        ],
        out_shape=[
            jax.ShapeDtypeStruct((N_NODES, D), jnp.float32),
            jax.ShapeDtypeStruct((N_NODES, D), jnp.float32),
        ],
    )(x, spec2d, wup, wsc)


def _gather_body(h_hbm, snd_hbm, out_hbm, idx_v, rows0, rows1, sem0, sem1):
    wid = lax.axis_index("s") * _NC + lax.axis_index("c")
    base = wid * _GPW
    pltpu.sync_copy(snd_hbm.at[wid], idx_v)

    def issue(t, buf, sem):
        return pltpu.async_copy(h_hbm.at[idx_v.at[pl.ds(t * _GCH, _GCH)]],
                                buf, sem)

    def wait(buf, sem):
        pltpu.make_async_copy(h_hbm.at[pl.ds(0, _GCH)], buf, sem).wait()

    def flush(t, buf):
        pltpu.sync_copy(buf, out_hbm.at[pl.ds(base + t * _GCH, _GCH)])

    issue(0, rows0, sem0)
    issue(1, rows1, sem1)

    def pair(p, carry):
        t0 = 2 * p
        wait(rows0, sem0)
        flush(t0, rows0)
        issue(t0 + 2, rows0, sem0)
        wait(rows1, sem1)
        flush(t0 + 1, rows1)
        issue(t0 + 3, rows1, sem1)
        return carry

    lax.fori_loop(0, (_GFULL - 3) // 2, pair, 0)
    t = _GFULL - 3
    wait(rows0, sem0)
    flush(t, rows0)
    issue(t + 2, rows0, sem0)
    wait(rows1, sem1)
    flush(t + 1, rows1)
    wait(rows0, sem0)
    flush(t + 2, rows0)
    pltpu.async_copy(
        h_hbm.at[idx_v.at[pl.ds(_GFULL * _GCH, _GTAIL)]],
        rows1.at[pl.ds(0, _GTAIL)], sem1).wait()
    pltpu.sync_copy(rows1.at[pl.ds(0, _GTAIL)],
                    out_hbm.at[pl.ds(base + _GFULL * _GCH, _GTAIL)])


def _gather_sc(h, snd):
    mesh = plsc.VectorSubcoreMesh(core_axis_name="c", subcore_axis_name="s",
                                  num_cores=_NC, num_subcores=_NS)
    f = functools.partial(
        pl.kernel,
        out_type=jax.ShapeDtypeStruct((N_EDGES, D), jnp.float32),
        mesh=mesh,
        compiler_params=pltpu.CompilerParams(use_tc_tiling_on_sc=False),
        scratch_types=[
            pltpu.VMEM((_GPW,), jnp.int32),
            pltpu.VMEM((_GCH, D), jnp.float32),
            pltpu.VMEM((_GCH, D), jnp.float32),
            pltpu.SemaphoreType.DMA,
            pltpu.SemaphoreType.DMA,
        ],
    )(_gather_body)
    return f(h, snd)


def _edge_body(vec_ref, msgs_ref, wtp_ref, p_ref, w1_ref, w2_ref, w3_ref,
               out_ref):
    vec = vec_ref[...]
    l2 = jnp.sum(vec * vec, axis=1, keepdims=True)
    length = jnp.sqrt(l2)
    inv = 1.0 / length
    ns = lax.broadcasted_iota(jnp.int32, (1, 8), 1).astype(jnp.float32) + 1.0
    radial = (np.float32(np.sqrt(2.0 / CUTOFF))
              * jnp.sin(ns * (np.pi / CUTOFF) * length) * inv)
    t = length * (1.0 / CUTOFF)
    t6 = t * t * t * t * t * t
    env = 1.0 - 28.0 * t6 + 48.0 * t6 * t - 21.0 * t6 * t * t
    env = jnp.where(t < 1.0, env, 0.0)
    radial = radial * env
    x = vec[:, 0:1] * inv
    y = vec[:, 1:2] * inv
    z = vec[:, 2:3] * inv
    x2, y2, z2 = x * x, y * y, z * z
    s3, s5, s7 = np.sqrt(3.0), np.sqrt(5.0), np.sqrt(7.0)
    s15, s105, s42, s70, s358 = (np.sqrt(15.0), np.sqrt(105.0),
                                 np.sqrt(42.0), np.sqrt(70.0),
                                 np.sqrt(35.0 / 8.0))
    sh = jnp.concatenate([
        s3 * x, s3 * y, s3 * z,
        s15 * x * y, s15 * y * z, (s5 / 2.0) * (3.0 * z2 - 1.0),
        s15 * x * z, (s15 / 2.0) * (x2 - y2),
        s358 * y * (3.0 * x2 - y2), s105 * x * y * z,
        (s42 / 4.0) * y * (5.0 * z2 - 1.0),
        (s7 / 2.0) * z * (5.0 * z2 - 3.0),
        (s42 / 4.0) * x * (5.0 * z2 - 1.0),
        (s105 / 2.0) * z * (x2 - y2),
        (s70 / 4.0) * x * (x2 - 3.0 * y2),
    ], axis=1)
    b1 = jnp.dot(sh, wtp_ref[...], preferred_element_type=jnp.float32)
    mix = _swish(jnp.dot(radial, w1_ref[...],
                         preferred_element_type=jnp.float32))
    mix = _swish(jnp.dot(mix, w2_ref[...],
                         preferred_element_type=jnp.float32))
    mix = jnp.dot(mix, w3_ref[...], preferred_element_type=jnp.float32)
    msgs = msgs_ref[...]
    p = p_ref[...]
    tp = jnp.dot(msgs[:, 0:NF], p,
                 preferred_element_type=jnp.float32) * b1[:, 0:D]
    for i in range(1, IR):
        tp = tp + jnp.dot(msgs[:, NF * i:NF * (i + 1)], p,
                          preferred_element_type=jnp.float32) \
            * b1[:, _IRP * i:_IRP * i + D]
    out_ref[...] = jnp.concatenate([msgs, tp], axis=1) * mix


def _edge_tc(vectors, msgs, wtpq, ptile, w1, w2, w3p):
    be = 800
    return pl.pallas_call(
        _edge_body,
        grid=(N_EDGES // be,),
        in_specs=[
            pl.BlockSpec((be, 3), lambda i: (i, 0)),
            pl.BlockSpec((be, D), lambda i: (i, 0)),
            pl.BlockSpec((SH_DIM, IR * _IRP), lambda i: (0, 0)),
            pl.BlockSpec((NF, D), lambda i: (0, 0)),
            pl.BlockSpec((8, 64), lambda i: (0, 0)),
            pl.BlockSpec((64, 64), lambda i: (0, 0)),
            pl.BlockSpec((64, MSG_DIM), lambda i: (0, 0)),
        ],
        out_specs=pl.BlockSpec((be, MSG_DIM), lambda i: (i, 0)),
        out_shape=jax.ShapeDtypeStruct((N_EDGES, MSG_DIM), jnp.float32),
    )(vectors, msgs, wtpq, ptile, w1, w2, w3p)


def _scatter_body(msg_hbm, recv_hbm, z_hbm, agg_hbm,
                  idx_v, buf0, buf1, stage, acc_sp, sem0, sem1):
    cid = lax.axis_index("c")
    sid = lax.axis_index("s")
    row0 = sid * _RPS
    pltpu.sync_copy(recv_hbm.at[sid], idx_v)

    for g_local in range(_GPC):
        col0 = (cid * _GPC + g_local) * _GRP
        pltpu.sync_copy(z_hbm, stage)
        for k in range(_RPS // _ZCH):
            pltpu.sync_copy(stage, acc_sp.at[pl.ds(row0 + k * _ZCH, _ZCH)])
        plsc.subcore_barrier()

        def load(tc, buf, sem):
            return pltpu.async_copy(
                msg_hbm.at[pl.ds(sid * _SPS + tc * _SCH, _SCH),
                           pl.ds(col0, _GRP)], buf, sem)

        def wait(buf, sem):
            pltpu.make_async_copy(
                msg_hbm.at[pl.ds(0, _SCH), pl.ds(0, _GRP)], buf, sem).wait()

        def scat(tc, buf):
            pltpu.sync_copy(buf, acc_sp.at[idx_v.at[tc]], add=True)

        load(0, buf0, sem0)
        load(1, buf1, sem1)

        def pair(p, carry):
            t0 = 2 * p
            wait(buf0, sem0)
            scat(t0, buf0)
            load(t0 + 2, buf0, sem0)
            wait(buf1, sem1)
            scat(t0 + 1, buf1)
            load(t0 + 3, buf1, sem1)
            return carry

        lax.fori_loop(0, (_SNCH - 3) // 2, pair, 0)
        t = _SNCH - 3
        wait(buf0, sem0)
        scat(t, buf0)
        load(t + 2, buf0, sem0)
        wait(buf1, sem1)
        scat(t + 1, buf1)
        wait(buf0, sem0)
        scat(t + 2, buf0)
        plsc.subcore_barrier()
        for k in range(_RPS // _ZCH):
            pltpu.sync_copy(acc_sp.at[pl.ds(row0 + k * _ZCH, _ZCH)], stage)
            pltpu.sync_copy(stage,
                            agg_hbm.at[pl.ds(row0 + k * _ZCH, _ZCH),
                                       pl.ds(col0, _GRP)])
        plsc.subcore_barrier()


def _scatter_sc(messages, rcv, zeros_grp):
    mesh = plsc.VectorSubcoreMesh(core_axis_name="c", subcore_axis_name="s",
                                  num_cores=_NC, num_subcores=_NS)
    f = functools.partial(
        pl.kernel,
        out_type=jax.ShapeDtypeStruct((N_NODES, MSG_DIM), jnp.float32),
        mesh=mesh,
        compiler_params=pltpu.CompilerParams(use_tc_tiling_on_sc=False),
        scratch_types=[
            pltpu.VMEM((_SNCH, _SCH), jnp.int32),
            pltpu.VMEM((_SCH, _GRP), jnp.float32),
            pltpu.VMEM((_SCH, _GRP), jnp.float32),
            pltpu.VMEM((_ZCH, _GRP), jnp.float32),
            pltpu.VMEM_SHARED((N_NODES, _GRP), jnp.float32),
            pltpu.SemaphoreType.DMA,
            pltpu.SemaphoreType.DMA,
        ],
    )(_scatter_body)
    return f(messages, rcv, zeros_grp)


def _final_body(agg_ref, sc_ref, wd_ref, r1_ref, r2_ref, out_ref):
    xg = jnp.dot(agg_ref[...], wd_ref[...], preferred_element_type=jnp.float32)
    s = xg[:, :NF]
    g1 = _swish(xg[:, NF:2 * NF])
    g2 = _swish(xg[:, 2 * NF:3 * NF])
    v1 = xg[:, 3 * NF:6 * NF]
    v2 = xg[:, 6 * NF:]
    s1 = jnp.dot(g1, r1_ref[...], preferred_element_type=jnp.float32)
    s2 = jnp.dot(g2, r2_ref[...], preferred_element_type=jnp.float32)
    out_ref[...] = jnp.concatenate([_swish(s), v1 * s1, v2 * s2],
                                   axis=1) + sc_ref[...]


def _final_tc(agg, sc, wd, r1, r2):
    bn = 2000
    return pl.pallas_call(
        _final_body,
        grid=(N_NODES // bn,),
        in_specs=[
            pl.BlockSpec((bn, MSG_DIM), lambda i: (i, 0)),
            pl.BlockSpec((bn, D), lambda i: (i, 0)),
            pl.BlockSpec((MSG_DIM, GATE_DIM), lambda i: (0, 0)),
            pl.BlockSpec((NF, 3 * NF), lambda i: (0, 0)),
            pl.BlockSpec((NF, 5 * NF), lambda i: (0, 0)),
        ],
        out_specs=pl.BlockSpec((bn, D), lambda i: (i, 0)),
        out_shape=jax.ShapeDtypeStruct((N_NODES, D), jnp.float32),
    )(agg, sc, wd, r1, r2)


def kernel(vectors, node_feats, node_specie, senders, receivers,
           W_sc, W_up, W_tp, W1, W2, W3, W_down):
    wup_p = W_up[:, _SIGMA]
    wtp_r = jnp.transpose(W_tp, (1, 0, 2)).reshape(SH_DIM, IR * IR)
    src = np.concatenate([np.clip(np.arange(_IRP) // NF, 0, IR - 1) + IR * i
                          for i in range(IR)])
    msk = np.concatenate([(np.arange(_IRP) < D).astype(np.float32)] * IR)
    wtpq = wtp_r[:, src] * msk[None, :]
    ptile = jnp.asarray(_P_TILE)
    w3_p = W3[:, _PI_FULL]
    wd_p = W_down[_PI_FULL, :] * np.float32(0.25)
    r1 = jnp.asarray(_R1)
    r2 = jnp.asarray(_R2)
    spec2d = node_specie.astype(jnp.int32).reshape(N_NODES, 1)
    snd = senders.astype(jnp.int32).reshape(_NW, _GPW)
    rcv = receivers.astype(jnp.int32).reshape(_NS, _SNCH, _SCH)
    zeros_grp = jnp.zeros((_ZCH, _GRP), jnp.float32)

    h, sc = _node_pre(node_feats, spec2d, wup_p, W_sc)
    msgs = _gather_sc(h, snd)
    messages = _edge_tc(vectors, msgs, wtpq, ptile, W1, W2, w3_p)
    agg = _scatter_sc(messages, rcv, zeros_grp)
    return _final_tc(agg, sc, wd_p, r1, r2)


# speedup vs baseline: 2.9697x; 1.2719x over previous
import functools

import numpy as np
import jax
import jax.numpy as jnp
from jax import lax
from jax.experimental import pallas as pl
from jax.experimental.pallas import tpu as pltpu
from jax.experimental.pallas import tpu_sc as plsc

N_NODES = 10000
N_EDGES = 160000
NF = 32
IR = 9
D = NF * IR
SH_DIM = 15
MSG_DIM = 2 * D
GATE_DIM = D + 2 * NF
CUTOFF = 3.0

_SIGMA = np.arange(D).reshape(NF, IR).T.reshape(-1)
_PI_FULL = np.concatenate([_SIGMA, D + _SIGMA])

_P_TILE = np.tile(np.eye(NF, dtype=np.float32), (1, IR))
_IRP = 384

_R1 = np.zeros((NF, 3 * NF), np.float32)
_R1[np.repeat(np.arange(NF), 3), np.arange(3 * NF)] = 1.0
_R2 = np.zeros((NF, 5 * NF), np.float32)
_R2[np.repeat(np.arange(NF), 5), np.arange(5 * NF)] = 1.0

_NC, _NS = 2, 16
_NW = _NC * _NS
_GPW = N_EDGES // _NW
_GCH = 128
_GFULL = _GPW // _GCH
_GTAIL = _GPW - _GFULL * _GCH
_SPS = N_EDGES // _NS
_SCH = 80
_SNCH = _SPS // _SCH
_GRP = MSG_DIM // 6
_GPC = 3
_RPS = N_NODES // _NS
_ZCH = 125


def _swish(x):
    return x / (1.0 + jnp.exp(-x))


def _node_pre_body(x_ref, spec_ref, wup_ref, wsc_ref, h_ref, sc_ref):
    x = x_ref[...]
    h_ref[...] = jnp.dot(x, wup_ref[...], preferred_element_type=jnp.float32)
    spec = spec_ref[...]
    acc = (spec == 0).astype(jnp.float32) * jnp.dot(
        x, wsc_ref[0], preferred_element_type=jnp.float32)
    for s in range(1, 5):
        acc = acc + (spec == s).astype(jnp.float32) * jnp.dot(
            x, wsc_ref[s], preferred_element_type=jnp.float32)
    sc_ref[...] = acc


def _node_pre(x, spec2d, wup, wsc):
    bn = 2000
    return pl.pallas_call(
        _node_pre_body,
        grid=(N_NODES // bn,),
        in_specs=[
            pl.BlockSpec((bn, D), lambda i: (i, 0)),
            pl.BlockSpec((bn, 1), lambda i: (i, 0)),
            pl.BlockSpec((D, D), lambda i: (0, 0)),
            pl.BlockSpec((5, D, D), lambda i: (0, 0, 0)),
        ],
        out_specs=[
            pl.BlockSpec((bn, D), lambda i: (i, 0)),
            pl.BlockSpec((bn, D), lambda i: (i, 0)),
        ],
        out_shape=[
            jax.ShapeDtypeStruct((N_NODES, D), jnp.float32),
            jax.ShapeDtypeStruct((N_NODES, D), jnp.float32),
        ],
    )(x, spec2d, wup, wsc)


def _gather_body(h_hbm, snd_hbm, out_hbm, idx_v, rows0, rows1, sem0, sem1):
    wid = lax.axis_index("s") * _NC + lax.axis_index("c")
    base = wid * _GPW
    pltpu.sync_copy(snd_hbm.at[wid], idx_v)

    def issue(t, buf, sem):
        return pltpu.async_copy(h_hbm.at[idx_v.at[pl.ds(t * _GCH, _GCH)]],
                                buf, sem)

    def wait(buf, sem):
        pltpu.make_async_copy(h_hbm.at[pl.ds(0, _GCH)], buf, sem).wait()

    def flush(t, buf):
        pltpu.sync_copy(buf, out_hbm.at[pl.ds(base + t * _GCH, _GCH)])

    issue(0, rows0, sem0)
    issue(1, rows1, sem1)

    def pair(p, carry):
        t0 = 2 * p
        wait(rows0, sem0)
        flush(t0, rows0)
        issue(t0 + 2, rows0, sem0)
        wait(rows1, sem1)
        flush(t0 + 1, rows1)
        issue(t0 + 3, rows1, sem1)
        return carry

    lax.fori_loop(0, (_GFULL - 3) // 2, pair, 0)
    t = _GFULL - 3
    wait(rows0, sem0)
    flush(t, rows0)
    issue(t + 2, rows0, sem0)
    wait(rows1, sem1)
    flush(t + 1, rows1)
    wait(rows0, sem0)
    flush(t + 2, rows0)
    pltpu.async_copy(
        h_hbm.at[idx_v.at[pl.ds(_GFULL * _GCH, _GTAIL)]],
        rows1.at[pl.ds(0, _GTAIL)], sem1).wait()
    pltpu.sync_copy(rows1.at[pl.ds(0, _GTAIL)],
                    out_hbm.at[pl.ds(base + _GFULL * _GCH, _GTAIL)])


def _gather_sc(h, snd):
    mesh = plsc.VectorSubcoreMesh(core_axis_name="c", subcore_axis_name="s",
                                  num_cores=_NC, num_subcores=_NS)
    f = functools.partial(
        pl.kernel,
        out_type=jax.ShapeDtypeStruct((N_EDGES, D), jnp.float32),
        mesh=mesh,
        compiler_params=pltpu.CompilerParams(use_tc_tiling_on_sc=False),
        scratch_types=[
            pltpu.VMEM((_GPW,), jnp.int32),
            pltpu.VMEM((_GCH, D), jnp.float32),
            pltpu.VMEM((_GCH, D), jnp.float32),
            pltpu.SemaphoreType.DMA,
            pltpu.SemaphoreType.DMA,
        ],
    )(_gather_body)
    return f(h, snd)


def _edge_body(vec_ref, msgs_ref, wtp_ref, p_ref, w1_ref, w2_ref, w3_ref,
               out_ref):
    vec = vec_ref[...]
    xr, yr, zr = vec[0:1, :], vec[1:2, :], vec[2:3, :]
    l2 = xr * xr + yr * yr + zr * zr
    length = jnp.sqrt(l2)
    inv = 1.0 / length
    ns = lax.broadcasted_iota(jnp.int32, (8, 1), 0).astype(jnp.float32) + 1.0
    t = length * (1.0 / CUTOFF)
    t6 = t * t * t
    t6 = t6 * t6
    env = 1.0 - 28.0 * t6 + 48.0 * t6 * t - 21.0 * t6 * t * t
    env = jnp.where(t < 1.0, env, 0.0)
    radial_t = (jnp.sin(ns * (np.pi / CUTOFF) * length)
                * (np.float32(np.sqrt(2.0 / CUTOFF)) * inv * env))
    x = xr * inv
    y = yr * inv
    z = zr * inv
    x2, y2, z2 = x * x, y * y, z * z
    s3, s5, s7 = np.sqrt(3.0), np.sqrt(5.0), np.sqrt(7.0)
    s15, s105, s42, s70, s358 = (np.sqrt(15.0), np.sqrt(105.0),
                                 np.sqrt(42.0), np.sqrt(70.0),
                                 np.sqrt(35.0 / 8.0))
    sh_t = jnp.concatenate([
        s3 * x, s3 * y, s3 * z,
        s15 * x * y, s15 * y * z, (s5 / 2.0) * (3.0 * z2 - 1.0),
        s15 * x * z, (s15 / 2.0) * (x2 - y2),
        s358 * y * (3.0 * x2 - y2), s105 * x * y * z,
        (s42 / 4.0) * y * (5.0 * z2 - 1.0),
        (s7 / 2.0) * z * (5.0 * z2 - 3.0),
        (s42 / 4.0) * x * (5.0 * z2 - 1.0),
        (s105 / 2.0) * z * (x2 - y2),
        (s70 / 4.0) * x * (x2 - 3.0 * y2),
    ], axis=0)
    b1 = lax.dot_general(sh_t, wtp_ref[...], (((0,), (0,)), ((), ())),
                         preferred_element_type=jnp.float32)
    mix = _swish(lax.dot_general(radial_t, w1_ref[...],
                                 (((0,), (0,)), ((), ())),
                                 preferred_element_type=jnp.float32))
    mix = _swish(jnp.dot(mix, w2_ref[...],
                         preferred_element_type=jnp.float32))
    mix = jnp.dot(mix, w3_ref[...], preferred_element_type=jnp.float32)
    msgs = msgs_ref[...]
    p = p_ref[...]
    tp = jnp.dot(msgs[:, 0:NF], p,
                 preferred_element_type=jnp.float32) * b1[:, 0:D]
    for i in range(1, IR):
        tp = tp + jnp.dot(msgs[:, NF * i:NF * (i + 1)], p,
                          preferred_element_type=jnp.float32) \
            * b1[:, _IRP * i:_IRP * i + D]
    out_ref[...] = jnp.concatenate([msgs, tp], axis=1) * mix


def _edge_tc(vec_t, msgs, wtpq, ptile, w1, w2, w3p):
    be = 640
    return pl.pallas_call(
        _edge_body,
        grid=(N_EDGES // be,),
        in_specs=[
            pl.BlockSpec((3, be), lambda i: (0, i)),
            pl.BlockSpec((be, D), lambda i: (i, 0)),
            pl.BlockSpec((SH_DIM, IR * _IRP), lambda i: (0, 0)),
            pl.BlockSpec((NF, D), lambda i: (0, 0)),
            pl.BlockSpec((8, 64), lambda i: (0, 0)),
            pl.BlockSpec((64, 64), lambda i: (0, 0)),
            pl.BlockSpec((64, MSG_DIM), lambda i: (0, 0)),
        ],
        out_specs=pl.BlockSpec((be, MSG_DIM), lambda i: (i, 0)),
        out_shape=jax.ShapeDtypeStruct((N_EDGES, MSG_DIM), jnp.float32),
    )(vec_t, msgs, wtpq, ptile, w1, w2, w3p)


def _scatter_body(msg_hbm, recv_hbm, z_hbm, agg_hbm,
                  idx_v, buf0, buf1, stage, acc_sp, sem0, sem1):
    cid = lax.axis_index("c")
    sid = lax.axis_index("s")
    row0 = sid * _RPS
    pltpu.sync_copy(recv_hbm.at[sid], idx_v)

    for g_local in range(_GPC):
        col0 = (cid * _GPC + g_local) * _GRP
        pltpu.sync_copy(z_hbm, stage)
        for k in range(_RPS // _ZCH):
            pltpu.sync_copy(stage, acc_sp.at[pl.ds(row0 + k * _ZCH, _ZCH)])
        plsc.subcore_barrier()

        def load(tc, buf, sem):
            return pltpu.async_copy(
                msg_hbm.at[pl.ds(sid * _SPS + tc * _SCH, _SCH),
                           pl.ds(col0, _GRP)], buf, sem)

        def wait(buf, sem):
            pltpu.make_async_copy(
                msg_hbm.at[pl.ds(0, _SCH), pl.ds(0, _GRP)], buf, sem).wait()

        def scat(tc, buf):
            pltpu.sync_copy(buf, acc_sp.at[idx_v.at[tc]], add=True)

        load(0, buf0, sem0)
        load(1, buf1, sem1)

        def pair(p, carry):
            t0 = 2 * p
            wait(buf0, sem0)
            scat(t0, buf0)
            load(t0 + 2, buf0, sem0)
            wait(buf1, sem1)
            scat(t0 + 1, buf1)
            load(t0 + 3, buf1, sem1)
            return carry

        lax.fori_loop(0, (_SNCH - 3) // 2, pair, 0)
        t = _SNCH - 3
        wait(buf0, sem0)
        scat(t, buf0)
        load(t + 2, buf0, sem0)
        wait(buf1, sem1)
        scat(t + 1, buf1)
        wait(buf0, sem0)
        scat(t + 2, buf0)
        plsc.subcore_barrier()
        for k in range(_RPS // _ZCH):
            pltpu.sync_copy(acc_sp.at[pl.ds(row0 + k * _ZCH, _ZCH)], stage)
            pltpu.sync_copy(stage,
                            agg_hbm.at[pl.ds(row0 + k * _ZCH, _ZCH),
                                       pl.ds(col0, _GRP)])
        plsc.subcore_barrier()


def _scatter_sc(messages, rcv, zeros_grp):
    mesh = plsc.VectorSubcoreMesh(core_axis_name="c", subcore_axis_name="s",
                                  num_cores=_NC, num_subcores=_NS)
    f = functools.partial(
        pl.kernel,
        out_type=jax.ShapeDtypeStruct((N_NODES, MSG_DIM), jnp.float32),
        mesh=mesh,
        compiler_params=pltpu.CompilerParams(use_tc_tiling_on_sc=False),
        scratch_types=[
            pltpu.VMEM((_SNCH, _SCH), jnp.int32),
            pltpu.VMEM((_SCH, _GRP), jnp.float32),
            pltpu.VMEM((_SCH, _GRP), jnp.float32),
            pltpu.VMEM((_ZCH, _GRP), jnp.float32),
            pltpu.VMEM_SHARED((N_NODES, _GRP), jnp.float32),
            pltpu.SemaphoreType.DMA,
            pltpu.SemaphoreType.DMA,
        ],
    )(_scatter_body)
    return f(messages, rcv, zeros_grp)


def _final_body(agg_ref, sc_ref, wd_ref, r1_ref, r2_ref, out_ref):
    xg = jnp.dot(agg_ref[...], wd_ref[...], preferred_element_type=jnp.float32)
    s = xg[:, :NF]
    g1 = _swish(xg[:, NF:2 * NF])
    g2 = _swish(xg[:, 2 * NF:3 * NF])
    v1 = xg[:, 3 * NF:6 * NF]
    v2 = xg[:, 6 * NF:]
    s1 = jnp.dot(g1, r1_ref[...], preferred_element_type=jnp.float32)
    s2 = jnp.dot(g2, r2_ref[...], preferred_element_type=jnp.float32)
    out_ref[...] = jnp.concatenate([_swish(s), v1 * s1, v2 * s2],
                                   axis=1) + sc_ref[...]


def _final_tc(agg, sc, wd, r1, r2):
    bn = 2000
    return pl.pallas_call(
        _final_body,
        grid=(N_NODES // bn,),
        in_specs=[
            pl.BlockSpec((bn, MSG_DIM), lambda i: (i, 0)),
            pl.BlockSpec((bn, D), lambda i: (i, 0)),
            pl.BlockSpec((MSG_DIM, GATE_DIM), lambda i: (0, 0)),
            pl.BlockSpec((NF, 3 * NF), lambda i: (0, 0)),
            pl.BlockSpec((NF, 5 * NF), lambda i: (0, 0)),
        ],
        out_specs=pl.BlockSpec((bn, D), lambda i: (i, 0)),
        out_shape=jax.ShapeDtypeStruct((N_NODES, D), jnp.float32),
    )(agg, sc, wd, r1, r2)


def kernel(vectors, node_feats, node_specie, senders, receivers,
           W_sc, W_up, W_tp, W1, W2, W3, W_down):
    wup_p = W_up[:, _SIGMA]
    wtp_r = jnp.transpose(W_tp, (1, 0, 2)).reshape(SH_DIM, IR * IR)
    src = np.concatenate([np.clip(np.arange(_IRP) // NF, 0, IR - 1) + IR * i
                          for i in range(IR)])
    msk = np.concatenate([(np.arange(_IRP) < D).astype(np.float32)] * IR)
    wtpq = wtp_r[:, src] * msk[None, :]
    ptile = jnp.asarray(_P_TILE)
    w3_p = W3[:, _PI_FULL]
    wd_p = W_down[_PI_FULL, :] * np.float32(0.25)
    r1 = jnp.asarray(_R1)
    r2 = jnp.asarray(_R2)
    spec2d = node_specie.astype(jnp.int32).reshape(N_NODES, 1)
    snd = senders.astype(jnp.int32).reshape(_NW, _GPW)
    rcv = receivers.astype(jnp.int32).reshape(_NS, _SNCH, _SCH)
    zeros_grp = jnp.zeros((_ZCH, _GRP), jnp.float32)

    h, sc = _node_pre(node_feats, spec2d, wup_p, W_sc)
    msgs = _gather_sc(h, snd)
    messages = _edge_tc(jnp.transpose(vectors), msgs, wtpq, ptile, W1, W2,
                        w3_p)
    agg = _scatter_sc(messages, rcv, zeros_grp)
    return _final_tc(agg, sc, wd_p, r1, r2)


# speedup vs baseline: 4.2020x; 1.4150x over previous
import functools

import numpy as np
import jax
import jax.numpy as jnp
from jax import lax
from jax.experimental import pallas as pl
from jax.experimental.pallas import tpu as pltpu
from jax.experimental.pallas import tpu_sc as plsc

N_NODES = 10000
N_EDGES = 160000
NF = 32
IR = 9
D = NF * IR
SH_DIM = 15
MSG_DIM = 2 * D
GATE_DIM = D + 2 * NF
CUTOFF = 3.0

_SIGMA = np.arange(D).reshape(NF, IR).T.reshape(-1)
_PI_FULL = np.concatenate([_SIGMA, D + _SIGMA])

_P_TILE = np.tile(np.eye(NF, dtype=np.float32), (1, IR))
_IRP = 384

_R1 = np.zeros((NF, 3 * NF), np.float32)
_R1[np.repeat(np.arange(NF), 3), np.arange(3 * NF)] = 1.0
_R2 = np.zeros((NF, 5 * NF), np.float32)
_R2[np.repeat(np.arange(NF), 5), np.arange(5 * NF)] = 1.0

_DP = 384
_MP = 640
_NC, _NS = 2, 16
_NW = _NC * _NS
_GPW = N_EDGES // _NW
_GCH = 128
_GFULL = _GPW // _GCH
_GTAIL = _GPW - _GFULL * _GCH
_SPS = N_EDGES // _NS
_SCH = 80
_SNCH = _SPS // _SCH
_GRP = 128
_NGRP = _MP // _GRP
_WBR = 640
_WCH = 80


def _swish(x):
    return x / (1.0 + jnp.exp(-x))


def _node_pre_body(x_ref, spec_ref, wup_ref, wsc_ref, h_ref, sc_ref):
    x = x_ref[...]
    h_ref[...] = jnp.dot(x, wup_ref[...], preferred_element_type=jnp.float32)
    spec = spec_ref[...]
    acc = (spec == 0).astype(jnp.float32) * jnp.dot(
        x, wsc_ref[0], preferred_element_type=jnp.float32)
    for s in range(1, 5):
        acc = acc + (spec == s).astype(jnp.float32) * jnp.dot(
            x, wsc_ref[s], preferred_element_type=jnp.float32)
    sc_ref[...] = acc


def _node_pre(x, spec2d, wup, wsc):
    bn = 2000
    return pl.pallas_call(
        _node_pre_body,
        grid=(N_NODES // bn,),
        in_specs=[
            pl.BlockSpec((bn, D), lambda i: (i, 0)),
            pl.BlockSpec((bn, 1), lambda i: (i, 0)),
            pl.BlockSpec((D, _DP), lambda i: (0, 0)),
            pl.BlockSpec((5, D, D), lambda i: (0, 0, 0)),
        ],
        out_specs=[
            pl.BlockSpec((bn, _DP), lambda i: (i, 0)),
            pl.BlockSpec((bn, D), lambda i: (i, 0)),
        ],
        out_shape=[
            jax.ShapeDtypeStruct((N_NODES, _DP), jnp.float32),
            jax.ShapeDtypeStruct((N_NODES, D), jnp.float32),
        ],
    )(x, spec2d, wup, wsc)


def _gather_body(h_hbm, snd_hbm, out_hbm, idx_v, rows0, rows1, sem0, sem1):
    wid = lax.axis_index("s") * _NC + lax.axis_index("c")
    base = wid * _GPW
    pltpu.sync_copy(snd_hbm.at[pl.ds(base, _GPW)], idx_v)

    def issue(t, buf, sem):
        return pltpu.async_copy(h_hbm.at[idx_v.at[pl.ds(t * _GCH, _GCH)]],
                                buf, sem)

    def wait(buf, sem):
        pltpu.make_async_copy(h_hbm.at[pl.ds(0, _GCH)], buf, sem).wait()

    def flush(t, buf):
        pltpu.sync_copy(buf, out_hbm.at[pl.ds(base + t * _GCH, _GCH)])

    issue(0, rows0, sem0)
    issue(1, rows1, sem1)

    def pair(p, carry):
        t0 = 2 * p
        wait(rows0, sem0)
        flush(t0, rows0)
        issue(t0 + 2, rows0, sem0)
        wait(rows1, sem1)
        flush(t0 + 1, rows1)
        issue(t0 + 3, rows1, sem1)
        return carry

    lax.fori_loop(0, (_GFULL - 3) // 2, pair, 0)
    t = _GFULL - 3
    wait(rows0, sem0)
    flush(t, rows0)
    issue(t + 2, rows0, sem0)
    wait(rows1, sem1)
    flush(t + 1, rows1)
    wait(rows0, sem0)
    flush(t + 2, rows0)
    pltpu.async_copy(
        h_hbm.at[idx_v.at[pl.ds(_GFULL * _GCH, _GTAIL)]],
        rows1.at[pl.ds(0, _GTAIL)], sem1).wait()
    pltpu.sync_copy(rows1.at[pl.ds(0, _GTAIL)],
                    out_hbm.at[pl.ds(base + _GFULL * _GCH, _GTAIL)])


def _gather_sc(h, snd):
    mesh = plsc.VectorSubcoreMesh(core_axis_name="c", subcore_axis_name="s",
                                  num_cores=_NC, num_subcores=_NS)
    f = functools.partial(
        pl.kernel,
        out_type=jax.ShapeDtypeStruct((N_EDGES, _DP), jnp.float32),
        mesh=mesh,
        scratch_types=[
            pltpu.VMEM((_GPW,), jnp.int32),
            pltpu.VMEM((_GCH, _DP), jnp.float32),
            pltpu.VMEM((_GCH, _DP), jnp.float32),
            pltpu.SemaphoreType.DMA,
            pltpu.SemaphoreType.DMA,
        ],
    )(_gather_body)
    return f(h, snd)


def _edge_body(vec_ref, msgs_ref, wtp_ref, p_ref, w1_ref, w2_ref, w3_ref,
               out_ref):
    vec = vec_ref[...]
    xr, yr, zr = vec[0:1, :], vec[1:2, :], vec[2:3, :]
    l2 = xr * xr + yr * yr + zr * zr
    length = jnp.sqrt(l2)
    inv = 1.0 / length
    ns = lax.broadcasted_iota(jnp.int32, (8, 1), 0).astype(jnp.float32) + 1.0
    t = length * (1.0 / CUTOFF)
    t6 = t * t * t
    t6 = t6 * t6
    env = 1.0 - 28.0 * t6 + 48.0 * t6 * t - 21.0 * t6 * t * t
    env = jnp.where(t < 1.0, env, 0.0)
    radial_t = (jnp.sin(ns * (np.pi / CUTOFF) * length)
                * (np.float32(np.sqrt(2.0 / CUTOFF)) * inv * env))
    x = xr * inv
    y = yr * inv
    z = zr * inv
    x2, y2, z2 = x * x, y * y, z * z
    s3, s5, s7 = np.sqrt(3.0), np.sqrt(5.0), np.sqrt(7.0)
    s15, s105, s42, s70, s358 = (np.sqrt(15.0), np.sqrt(105.0),
                                 np.sqrt(42.0), np.sqrt(70.0),
                                 np.sqrt(35.0 / 8.0))
    sh_t = jnp.concatenate([
        s3 * x, s3 * y, s3 * z,
        s15 * x * y, s15 * y * z, (s5 / 2.0) * (3.0 * z2 - 1.0),
        s15 * x * z, (s15 / 2.0) * (x2 - y2),
        s358 * y * (3.0 * x2 - y2), s105 * x * y * z,
        (s42 / 4.0) * y * (5.0 * z2 - 1.0),
        (s7 / 2.0) * z * (5.0 * z2 - 3.0),
        (s42 / 4.0) * x * (5.0 * z2 - 1.0),
        (s105 / 2.0) * z * (x2 - y2),
        (s70 / 4.0) * x * (x2 - 3.0 * y2),
    ], axis=0)
    b1 = lax.dot_general(sh_t, wtp_ref[...], (((0,), (0,)), ((), ())),
                         preferred_element_type=jnp.float32)
    mix = _swish(lax.dot_general(radial_t, w1_ref[...],
                                 (((0,), (0,)), ((), ())),
                                 preferred_element_type=jnp.float32))
    mix = _swish(jnp.dot(mix, w2_ref[...],
                         preferred_element_type=jnp.float32))
    mix = jnp.dot(mix, w3_ref[...], preferred_element_type=jnp.float32)
    msgs = msgs_ref[...][:, :D]
    p = p_ref[...]
    tp = jnp.dot(msgs[:, 0:NF], p,
                 preferred_element_type=jnp.float32) * b1[:, 0:D]
    for i in range(1, IR):
        tp = tp + jnp.dot(msgs[:, NF * i:NF * (i + 1)], p,
                          preferred_element_type=jnp.float32) \
            * b1[:, _IRP * i:_IRP * i + D]
    be = msgs.shape[0]
    out_ref[...] = jnp.concatenate(
        [jnp.concatenate([msgs, tp], axis=1) * mix,
         jnp.zeros((be, _MP - MSG_DIM), jnp.float32)], axis=1)


def _edge_tc(vec_t, msgs, wtpq, ptile, w1, w2, w3p):
    be = 640
    return pl.pallas_call(
        _edge_body,
        grid=(N_EDGES // be,),
        in_specs=[
            pl.BlockSpec((3, be), lambda i: (0, i)),
            pl.BlockSpec((be, _DP), lambda i: (i, 0)),
            pl.BlockSpec((SH_DIM, IR * _IRP), lambda i: (0, 0)),
            pl.BlockSpec((NF, D), lambda i: (0, 0)),
            pl.BlockSpec((8, 64), lambda i: (0, 0)),
            pl.BlockSpec((64, 64), lambda i: (0, 0)),
            pl.BlockSpec((64, MSG_DIM), lambda i: (0, 0)),
        ],
        out_specs=pl.BlockSpec((be, _MP), lambda i: (i, 0)),
        out_shape=jax.ShapeDtypeStruct((N_EDGES, _MP), jnp.float32),
    )(vec_t, msgs, wtpq, ptile, w1, w2, w3p)


def _scatter_body(msg_hbm, recv_hbm, z_hbm, agg_hbm,
                  idx_v, buf0, buf1, stage, acc_sp, sem0, sem1):
    cid = lax.axis_index("c")
    sid = lax.axis_index("s")
    row0 = sid * _WBR
    nch = jnp.where(sid == _NS - 1, (N_NODES - (_NS - 1) * _WBR) // _WCH,
                    _WBR // _WCH)
    pltpu.sync_copy(recv_hbm.at[sid], idx_v)

    for g_local in range(3):
        col0 = jnp.minimum(cid * 3 + g_local, _NGRP - 1) * _GRP
        pltpu.sync_copy(z_hbm, stage)

        def zero(k, carry):
            pltpu.sync_copy(stage, acc_sp.at[pl.ds(row0 + k * _WCH, _WCH)])
            return carry

        lax.fori_loop(0, nch, zero, 0)
        plsc.subcore_barrier()

        def load(tc, buf, sem):
            return pltpu.async_copy(
                msg_hbm.at[pl.ds(sid * _SPS + tc * _SCH, _SCH),
                           pl.ds(col0, _GRP)], buf, sem)

        def wait(buf, sem):
            pltpu.make_async_copy(
                msg_hbm.at[pl.ds(0, _SCH), pl.ds(0, _GRP)], buf, sem).wait()

        def scat(tc, buf):
            pltpu.sync_copy(buf, acc_sp.at[idx_v.at[tc]], add=True)

        load(0, buf0, sem0)
        load(1, buf1, sem1)

        def pair(p, carry):
            t0 = 2 * p
            wait(buf0, sem0)
            scat(t0, buf0)
            load(t0 + 2, buf0, sem0)
            wait(buf1, sem1)
            scat(t0 + 1, buf1)
            load(t0 + 3, buf1, sem1)
            return carry

        lax.fori_loop(0, (_SNCH - 3) // 2, pair, 0)
        t = _SNCH - 3
        wait(buf0, sem0)
        scat(t, buf0)
        load(t + 2, buf0, sem0)
        wait(buf1, sem1)
        scat(t + 1, buf1)
        wait(buf0, sem0)
        scat(t + 2, buf0)
        plsc.subcore_barrier()

        def wb(k, carry):
            pltpu.sync_copy(acc_sp.at[pl.ds(row0 + k * _WCH, _WCH)], stage)
            pltpu.sync_copy(stage,
                            agg_hbm.at[pl.ds(row0 + k * _WCH, _WCH),
                                       pl.ds(col0, _GRP)])
            return carry

        lax.fori_loop(0, nch, wb, 0)
        plsc.subcore_barrier()


def _scatter_sc(messages, rcv, zeros_grp):
    mesh = plsc.VectorSubcoreMesh(core_axis_name="c", subcore_axis_name="s",
                                  num_cores=_NC, num_subcores=_NS)
    f = functools.partial(
        pl.kernel,
        out_type=jax.ShapeDtypeStruct((N_NODES, _MP), jnp.float32),
        mesh=mesh,
        scratch_types=[
            pltpu.VMEM((_SNCH, _SCH), jnp.int32),
            pltpu.VMEM((_SCH, _GRP), jnp.float32),
            pltpu.VMEM((_SCH, _GRP), jnp.float32),
            pltpu.VMEM((_WCH, _GRP), jnp.float32),
            pltpu.VMEM_SHARED((N_NODES, _GRP), jnp.float32),
            pltpu.SemaphoreType.DMA,
            pltpu.SemaphoreType.DMA,
        ],
    )(_scatter_body)
    return f(messages, rcv, zeros_grp)


def _final_body(agg_ref, sc_ref, wd_ref, r1_ref, r2_ref, out_ref):
    xg = jnp.dot(agg_ref[...][:, :MSG_DIM], wd_ref[...],
                 preferred_element_type=jnp.float32)
    s = xg[:, :NF]
    g1 = _swish(xg[:, NF:2 * NF])
    g2 = _swish(xg[:, 2 * NF:3 * NF])
    v1 = xg[:, 3 * NF:6 * NF]
    v2 = xg[:, 6 * NF:]
    s1 = jnp.dot(g1, r1_ref[...], preferred_element_type=jnp.float32)
    s2 = jnp.dot(g2, r2_ref[...], preferred_element_type=jnp.float32)
    out_ref[...] = jnp.concatenate([_swish(s), v1 * s1, v2 * s2],
                                   axis=1) + sc_ref[...]


def _final_tc(agg, sc, wd, r1, r2):
    bn = 2000
    return pl.pallas_call(
        _final_body,
        grid=(N_NODES // bn,),
        in_specs=[
            pl.BlockSpec((bn, _MP), lambda i: (i, 0)),
            pl.BlockSpec((bn, D), lambda i: (i, 0)),
            pl.BlockSpec((MSG_DIM, GATE_DIM), lambda i: (0, 0)),
            pl.BlockSpec((NF, 3 * NF), lambda i: (0, 0)),
            pl.BlockSpec((NF, 5 * NF), lambda i: (0, 0)),
        ],
        out_specs=pl.BlockSpec((bn, D), lambda i: (i, 0)),
        out_shape=jax.ShapeDtypeStruct((N_NODES, D), jnp.float32),
    )(agg, sc, wd, r1, r2)


def kernel(vectors, node_feats, node_specie, senders, receivers,
           W_sc, W_up, W_tp, W1, W2, W3, W_down):
    wup_p = jnp.pad(W_up[:, _SIGMA], ((0, 0), (0, _DP - D)))
    wtp_r = jnp.transpose(W_tp, (1, 0, 2)).reshape(SH_DIM, IR * IR)
    src = np.concatenate([np.clip(np.arange(_IRP) // NF, 0, IR - 1) + IR * i
                          for i in range(IR)])
    msk = np.concatenate([(np.arange(_IRP) < D).astype(np.float32)] * IR)
    wtpq = wtp_r[:, src] * msk[None, :]
    ptile = jnp.asarray(_P_TILE)
    w3_p = W3[:, _PI_FULL]
    wd_p = W_down[_PI_FULL, :] * np.float32(0.25)
    r1 = jnp.asarray(_R1)
    r2 = jnp.asarray(_R2)
    spec2d = node_specie.astype(jnp.int32).reshape(N_NODES, 1)
    snd = senders.astype(jnp.int32)
    rcv = receivers.astype(jnp.int32).reshape(_NS, _SNCH, _SCH)
    zeros_grp = jnp.zeros((_WCH, _GRP), jnp.float32)

    h, sc = _node_pre(node_feats, spec2d, wup_p, W_sc)
    msgs = _gather_sc(h, snd)
    messages = _edge_tc(jnp.transpose(vectors), msgs, wtpq, ptile, W1, W2,
                        w3_p)
    agg = _scatter_sc(messages, rcv, zeros_grp)
    return _final_tc(agg, sc, wd_p, r1, r2)


# speedup vs baseline: 4.2194x; 1.0041x over previous
import functools

import numpy as np
import jax
import jax.numpy as jnp
from jax import lax
from jax.experimental import pallas as pl
from jax.experimental.pallas import tpu as pltpu
from jax.experimental.pallas import tpu_sc as plsc

N_NODES = 10000
N_EDGES = 160000
NF = 32
IR = 9
D = NF * IR
SH_DIM = 15
MSG_DIM = 2 * D
GATE_DIM = D + 2 * NF
CUTOFF = 3.0

_SIGMA = np.arange(D).reshape(NF, IR).T.reshape(-1)
_PI_FULL = np.concatenate([_SIGMA, D + _SIGMA])

_P_TILE = np.tile(np.eye(NF, dtype=np.float32), (1, IR))
_IRP = 384

_R1 = np.zeros((NF, 3 * NF), np.float32)
_R1[np.repeat(np.arange(NF), 3), np.arange(3 * NF)] = 1.0
_R2 = np.zeros((NF, 5 * NF), np.float32)
_R2[np.repeat(np.arange(NF), 5), np.arange(5 * NF)] = 1.0

_DP = 384
_MP = 640
_NC, _NS = 2, 16
_NW = _NC * _NS
_GPW = N_EDGES // _NW
_GCH = 128
_GFULL = _GPW // _GCH
_GTAIL = _GPW - _GFULL * _GCH
_SPS = N_EDGES // _NS
_SCH = 80
_SNCH = _SPS // _SCH
_GRP = 128
_NGRP = _MP // _GRP
_WBR = 640
_WCH = 80


def _swish(x):
    return x / (1.0 + jnp.exp(-x))


def _node_pre_body(x_ref, spec_ref, wup_ref, wsc_ref, h_ref, sc_ref):
    x = x_ref[...]
    h_ref[...] = jnp.dot(x, wup_ref[...], preferred_element_type=jnp.float32)
    spec = spec_ref[...]
    acc = (spec == 0).astype(jnp.float32) * jnp.dot(
        x, wsc_ref[0], preferred_element_type=jnp.float32)
    for s in range(1, 5):
        acc = acc + (spec == s).astype(jnp.float32) * jnp.dot(
            x, wsc_ref[s], preferred_element_type=jnp.float32)
    sc_ref[...] = acc


def _node_pre(x, spec2d, wup, wsc):
    bn = 2000
    return pl.pallas_call(
        _node_pre_body,
        grid=(N_NODES // bn,),
        in_specs=[
            pl.BlockSpec((bn, D), lambda i: (i, 0)),
            pl.BlockSpec((bn, 1), lambda i: (i, 0)),
            pl.BlockSpec((D, _DP), lambda i: (0, 0)),
            pl.BlockSpec((5, D, D), lambda i: (0, 0, 0)),
        ],
        out_specs=[
            pl.BlockSpec((bn, _DP), lambda i: (i, 0)),
            pl.BlockSpec((bn, D), lambda i: (i, 0)),
        ],
        out_shape=[
            jax.ShapeDtypeStruct((N_NODES, _DP), jnp.float32),
            jax.ShapeDtypeStruct((N_NODES, D), jnp.float32),
        ],
    )(x, spec2d, wup, wsc)


def _gather_body(h_hbm, snd_hbm, out_hbm, idx_v, rows0, rows1, sem0, sem1):
    wid = lax.axis_index("s") * _NC + lax.axis_index("c")
    base = wid * _GPW
    pltpu.sync_copy(snd_hbm.at[pl.ds(base, _GPW)], idx_v)

    def issue(t, buf, sem):
        return pltpu.async_copy(h_hbm.at[idx_v.at[pl.ds(t * _GCH, _GCH)]],
                                buf, sem)

    def wait(buf, sem):
        pltpu.make_async_copy(h_hbm.at[pl.ds(0, _GCH)], buf, sem).wait()

    def flush(t, buf):
        pltpu.sync_copy(buf, out_hbm.at[pl.ds(base + t * _GCH, _GCH)])

    issue(0, rows0, sem0)
    issue(1, rows1, sem1)

    def pair(p, carry):
        t0 = 2 * p
        wait(rows0, sem0)
        flush(t0, rows0)
        issue(t0 + 2, rows0, sem0)
        wait(rows1, sem1)
        flush(t0 + 1, rows1)
        issue(t0 + 3, rows1, sem1)
        return carry

    lax.fori_loop(0, (_GFULL - 3) // 2, pair, 0)
    t = _GFULL - 3
    wait(rows0, sem0)
    flush(t, rows0)
    issue(t + 2, rows0, sem0)
    wait(rows1, sem1)
    flush(t + 1, rows1)
    wait(rows0, sem0)
    flush(t + 2, rows0)
    pltpu.async_copy(
        h_hbm.at[idx_v.at[pl.ds(_GFULL * _GCH, _GTAIL)]],
        rows1.at[pl.ds(0, _GTAIL)], sem1).wait()
    pltpu.sync_copy(rows1.at[pl.ds(0, _GTAIL)],
                    out_hbm.at[pl.ds(base + _GFULL * _GCH, _GTAIL)])


def _gather_sc(h, snd):
    mesh = plsc.VectorSubcoreMesh(core_axis_name="c", subcore_axis_name="s",
                                  num_cores=_NC, num_subcores=_NS)
    f = functools.partial(
        pl.kernel,
        out_type=jax.ShapeDtypeStruct((N_EDGES, _DP), jnp.float32),
        mesh=mesh,
        scratch_types=[
            pltpu.VMEM((_GPW,), jnp.int32),
            pltpu.VMEM((_GCH, _DP), jnp.float32),
            pltpu.VMEM((_GCH, _DP), jnp.float32),
            pltpu.SemaphoreType.DMA,
            pltpu.SemaphoreType.DMA,
        ],
    )(_gather_body)
    return f(h, snd)


def _edge_body(vec_ref, msgs_ref, wtp_ref, p_ref, w1_ref, w2_ref, w3_ref,
               out_ref):
    vec = vec_ref[...]
    xr, yr, zr = vec[0:1, :], vec[1:2, :], vec[2:3, :]
    l2 = xr * xr + yr * yr + zr * zr
    length = jnp.sqrt(l2)
    inv = 1.0 / length
    ns = lax.broadcasted_iota(jnp.int32, (8, 1), 0).astype(jnp.float32) + 1.0
    t = length * (1.0 / CUTOFF)
    t6 = t * t * t
    t6 = t6 * t6
    env = 1.0 - 28.0 * t6 + 48.0 * t6 * t - 21.0 * t6 * t * t
    env = jnp.where(t < 1.0, env, 0.0)
    radial_t = (jnp.sin(ns * (np.pi / CUTOFF) * length)
                * (np.float32(np.sqrt(2.0 / CUTOFF)) * inv * env))
    x = xr * inv
    y = yr * inv
    z = zr * inv
    x2, y2, z2 = x * x, y * y, z * z
    s3, s5, s7 = np.sqrt(3.0), np.sqrt(5.0), np.sqrt(7.0)
    s15, s105, s42, s70, s358 = (np.sqrt(15.0), np.sqrt(105.0),
                                 np.sqrt(42.0), np.sqrt(70.0),
                                 np.sqrt(35.0 / 8.0))
    sh_t = jnp.concatenate([
        s3 * x, s3 * y, s3 * z,
        s15 * x * y, s15 * y * z, (s5 / 2.0) * (3.0 * z2 - 1.0),
        s15 * x * z, (s15 / 2.0) * (x2 - y2),
        s358 * y * (3.0 * x2 - y2), s105 * x * y * z,
        (s42 / 4.0) * y * (5.0 * z2 - 1.0),
        (s7 / 2.0) * z * (5.0 * z2 - 3.0),
        (s42 / 4.0) * x * (5.0 * z2 - 1.0),
        (s105 / 2.0) * z * (x2 - y2),
        (s70 / 4.0) * x * (x2 - 3.0 * y2),
    ], axis=0)
    b1 = lax.dot_general(sh_t.astype(jnp.bfloat16), wtp_ref[...],
                         (((0,), (0,)), ((), ())),
                         preferred_element_type=jnp.float32)
    mix = _swish(lax.dot_general(radial_t, w1_ref[...],
                                 (((0,), (0,)), ((), ())),
                                 preferred_element_type=jnp.float32))
    mix = _swish(jnp.dot(mix, w2_ref[...],
                         preferred_element_type=jnp.float32))
    mix = jnp.dot(mix.astype(jnp.bfloat16), w3_ref[...],
                  preferred_element_type=jnp.float32)
    msgs = msgs_ref[...][:, :D]
    msgs_bf = msgs.astype(jnp.bfloat16)
    p = p_ref[...]
    tp = jnp.dot(msgs_bf[:, 0:NF], p,
                 preferred_element_type=jnp.float32) * b1[:, 0:D]
    for i in range(1, IR):
        tp = tp + jnp.dot(msgs_bf[:, NF * i:NF * (i + 1)], p,
                          preferred_element_type=jnp.float32) \
            * b1[:, _IRP * i:_IRP * i + D]
    be = msgs.shape[0]
    out_ref[...] = jnp.concatenate(
        [jnp.concatenate([msgs, tp], axis=1) * mix,
         jnp.zeros((be, _MP - MSG_DIM), jnp.float32)], axis=1)


def _edge_tc(vec_t, msgs, wtpq, ptile, w1, w2, w3p):
    be = 640
    return pl.pallas_call(
        _edge_body,
        grid=(N_EDGES // be,),
        in_specs=[
            pl.BlockSpec((3, be), lambda i: (0, i)),
            pl.BlockSpec((be, _DP), lambda i: (i, 0)),
            pl.BlockSpec((SH_DIM, IR * _IRP), lambda i: (0, 0)),
            pl.BlockSpec((NF, D), lambda i: (0, 0)),
            pl.BlockSpec((8, 64), lambda i: (0, 0)),
            pl.BlockSpec((64, 64), lambda i: (0, 0)),
            pl.BlockSpec((64, MSG_DIM), lambda i: (0, 0)),
        ],
        out_specs=pl.BlockSpec((be, _MP), lambda i: (i, 0)),
        out_shape=jax.ShapeDtypeStruct((N_EDGES, _MP), jnp.float32),
    )(vec_t, msgs, wtpq, ptile, w1, w2, w3p)


def _scatter_body(msg_hbm, recv_hbm, z_hbm, agg_hbm,
                  idx_v, buf0, buf1, stage, acc_sp, sem0, sem1):
    cid = lax.axis_index("c")
    sid = lax.axis_index("s")
    row0 = sid * _WBR
    nch = jnp.where(sid == _NS - 1, (N_NODES - (_NS - 1) * _WBR) // _WCH,
                    _WBR // _WCH)
    pltpu.sync_copy(recv_hbm.at[sid], idx_v)

    for g_local in range(3):
        col0 = jnp.minimum(cid * 3 + g_local, _NGRP - 1) * _GRP
        pltpu.sync_copy(z_hbm, stage)

        def zero(k, carry):
            pltpu.sync_copy(stage, acc_sp.at[pl.ds(row0 + k * _WCH, _WCH)])
            return carry

        lax.fori_loop(0, nch, zero, 0)
        plsc.subcore_barrier()

        def load(tc, buf, sem):
            return pltpu.async_copy(
                msg_hbm.at[pl.ds(sid * _SPS + tc * _SCH, _SCH),
                           pl.ds(col0, _GRP)], buf, sem)

        def wait(buf, sem):
            pltpu.make_async_copy(
                msg_hbm.at[pl.ds(0, _SCH), pl.ds(0, _GRP)], buf, sem).wait()

        def scat(tc, buf):
            pltpu.sync_copy(buf, acc_sp.at[idx_v.at[tc]], add=True)

        load(0, buf0, sem0)
        load(1, buf1, sem1)

        def pair(p, carry):
            t0 = 2 * p
            wait(buf0, sem0)
            scat(t0, buf0)
            load(t0 + 2, buf0, sem0)
            wait(buf1, sem1)
            scat(t0 + 1, buf1)
            load(t0 + 3, buf1, sem1)
            return carry

        lax.fori_loop(0, (_SNCH - 3) // 2, pair, 0)
        t = _SNCH - 3
        wait(buf0, sem0)
        scat(t, buf0)
        load(t + 2, buf0, sem0)
        wait(buf1, sem1)
        scat(t + 1, buf1)
        wait(buf0, sem0)
        scat(t + 2, buf0)
        plsc.subcore_barrier()

        def wb(k, carry):
            pltpu.sync_copy(acc_sp.at[pl.ds(row0 + k * _WCH, _WCH)], stage)
            pltpu.sync_copy(stage,
                            agg_hbm.at[pl.ds(row0 + k * _WCH, _WCH),
                                       pl.ds(col0, _GRP)])
            return carry

        lax.fori_loop(0, nch, wb, 0)
        plsc.subcore_barrier()


def _scatter_sc(messages, rcv, zeros_grp):
    mesh = plsc.VectorSubcoreMesh(core_axis_name="c", subcore_axis_name="s",
                                  num_cores=_NC, num_subcores=_NS)
    f = functools.partial(
        pl.kernel,
        out_type=jax.ShapeDtypeStruct((N_NODES, _MP), jnp.float32),
        mesh=mesh,
        scratch_types=[
            pltpu.VMEM((_SNCH, _SCH), jnp.int32),
            pltpu.VMEM((_SCH, _GRP), jnp.float32),
            pltpu.VMEM((_SCH, _GRP), jnp.float32),
            pltpu.VMEM((_WCH, _GRP), jnp.float32),
            pltpu.VMEM_SHARED((N_NODES, _GRP), jnp.float32),
            pltpu.SemaphoreType.DMA,
            pltpu.SemaphoreType.DMA,
        ],
    )(_scatter_body)
    return f(messages, rcv, zeros_grp)


def _final_body(agg_ref, sc_ref, wd_ref, r1_ref, r2_ref, out_ref):
    xg = jnp.dot(agg_ref[...][:, :MSG_DIM], wd_ref[...],
                 preferred_element_type=jnp.float32)
    s = xg[:, :NF]
    g1 = _swish(xg[:, NF:2 * NF])
    g2 = _swish(xg[:, 2 * NF:3 * NF])
    v1 = xg[:, 3 * NF:6 * NF]
    v2 = xg[:, 6 * NF:]
    s1 = jnp.dot(g1, r1_ref[...], preferred_element_type=jnp.float32)
    s2 = jnp.dot(g2, r2_ref[...], preferred_element_type=jnp.float32)
    out_ref[...] = jnp.concatenate([_swish(s), v1 * s1, v2 * s2],
                                   axis=1) + sc_ref[...]


def _final_tc(agg, sc, wd, r1, r2):
    bn = 2000
    return pl.pallas_call(
        _final_body,
        grid=(N_NODES // bn,),
        in_specs=[
            pl.BlockSpec((bn, _MP), lambda i: (i, 0)),
            pl.BlockSpec((bn, D), lambda i: (i, 0)),
            pl.BlockSpec((MSG_DIM, GATE_DIM), lambda i: (0, 0)),
            pl.BlockSpec((NF, 3 * NF), lambda i: (0, 0)),
            pl.BlockSpec((NF, 5 * NF), lambda i: (0, 0)),
        ],
        out_specs=pl.BlockSpec((bn, D), lambda i: (i, 0)),
        out_shape=jax.ShapeDtypeStruct((N_NODES, D), jnp.float32),
    )(agg, sc, wd, r1, r2)


def kernel(vectors, node_feats, node_specie, senders, receivers,
           W_sc, W_up, W_tp, W1, W2, W3, W_down):
    wup_p = jnp.pad(W_up[:, _SIGMA], ((0, 0), (0, _DP - D)))
    wtp_r = jnp.transpose(W_tp, (1, 0, 2)).reshape(SH_DIM, IR * IR)
    src = np.concatenate([np.clip(np.arange(_IRP) // NF, 0, IR - 1) + IR * i
                          for i in range(IR)])
    msk = np.concatenate([(np.arange(_IRP) < D).astype(np.float32)] * IR)
    wtpq = (wtp_r[:, src] * msk[None, :]).astype(jnp.bfloat16)
    ptile = jnp.asarray(_P_TILE).astype(jnp.bfloat16)
    w3_p = W3[:, _PI_FULL].astype(jnp.bfloat16)
    wd_p = W_down[_PI_FULL, :] * np.float32(0.25)
    r1 = jnp.asarray(_R1)
    r2 = jnp.asarray(_R2)
    spec2d = node_specie.astype(jnp.int32).reshape(N_NODES, 1)
    snd = senders.astype(jnp.int32)
    rcv = receivers.astype(jnp.int32).reshape(_NS, _SNCH, _SCH)
    zeros_grp = jnp.zeros((_WCH, _GRP), jnp.float32)

    h, sc = _node_pre(node_feats, spec2d, wup_p, W_sc)
    msgs = _gather_sc(h, snd)
    messages = _edge_tc(jnp.transpose(vectors), msgs, wtpq, ptile, W1, W2,
                        w3_p)
    agg = _scatter_sc(messages, rcv, zeros_grp)
    return _final_tc(agg, sc, wd_p, r1, r2)


# speedup vs baseline: 5.0009x; 1.1852x over previous
import functools

import numpy as np
import jax
import jax.numpy as jnp
from jax import lax
from jax.experimental import pallas as pl
from jax.experimental.pallas import tpu as pltpu
from jax.experimental.pallas import tpu_sc as plsc

N_NODES = 10000
N_EDGES = 160000
NF = 32
IR = 9
D = NF * IR
SH_DIM = 15
MSG_DIM = 2 * D
GATE_DIM = D + 2 * NF
CUTOFF = 3.0

_SIGMA = np.arange(D).reshape(NF, IR).T.reshape(-1)
_PI_FULL = np.concatenate([_SIGMA, D + _SIGMA])

_P_TILE = np.tile(np.eye(NF, dtype=np.float32), (1, IR))
_IRP = 384

_R1 = np.zeros((NF, 3 * NF), np.float32)
_R1[np.repeat(np.arange(NF), 3), np.arange(3 * NF)] = 1.0
_R2 = np.zeros((NF, 5 * NF), np.float32)
_R2[np.repeat(np.arange(NF), 5), np.arange(5 * NF)] = 1.0

_DP = 384
_MP = 640
_NC, _NS = 2, 16
_NW = _NC * _NS
_GPW = N_EDGES // _NW
_GCH = 128
_GFULL = _GPW // _GCH
_GTAIL = _GPW - _GFULL * _GCH
_SPS = N_EDGES // _NS
_SCH = 80
_SNCH = _SPS // _SCH
_GRP = 128
_ESPL = 96000
_NGRP = _MP // _GRP
_WBR = 640
_WCH = 80


def _swish(x):
    return x / (1.0 + jnp.exp(-x))


def _node_pre_body(x_ref, spec_ref, wup_ref, wsc_ref, h_ref, sc_ref):
    x = x_ref[...]
    h_ref[...] = jnp.dot(x, wup_ref[...], preferred_element_type=jnp.float32)
    spec = spec_ref[...]
    acc = (spec == 0).astype(jnp.float32) * jnp.dot(
        x, wsc_ref[0], preferred_element_type=jnp.float32)
    for s in range(1, 5):
        acc = acc + (spec == s).astype(jnp.float32) * jnp.dot(
            x, wsc_ref[s], preferred_element_type=jnp.float32)
    sc_ref[...] = acc


def _node_pre(x, spec2d, wup, wsc):
    bn = 2000
    return pl.pallas_call(
        _node_pre_body,
        grid=(N_NODES // bn,),
        in_specs=[
            pl.BlockSpec((bn, D), lambda i: (i, 0)),
            pl.BlockSpec((bn, 1), lambda i: (i, 0)),
            pl.BlockSpec((D, _DP), lambda i: (0, 0)),
            pl.BlockSpec((5, D, D), lambda i: (0, 0, 0)),
        ],
        out_specs=[
            pl.BlockSpec((bn, _DP), lambda i: (i, 0)),
            pl.BlockSpec((bn, D), lambda i: (i, 0)),
        ],
        out_shape=[
            jax.ShapeDtypeStruct((N_NODES, _DP), jnp.float32),
            jax.ShapeDtypeStruct((N_NODES, D), jnp.float32),
        ],
    )(x, spec2d, wup, wsc)


def _make_gather_body(rows_pw):
    nfull = rows_pw // _GCH
    tail = rows_pw - nfull * _GCH
    npair = (nfull - 2) // 2

    def body(h_hbm, snd_hbm, out_hbm, idx_v, rows0, rows1, sem0, sem1):
        wid = lax.axis_index("s") * _NC + lax.axis_index("c")
        base = wid * rows_pw
        pltpu.sync_copy(snd_hbm.at[pl.ds(base, rows_pw)], idx_v)
        bufs, sems = (rows0, rows1), (sem0, sem1)

        def issue(t, buf, sem):
            return pltpu.async_copy(h_hbm.at[idx_v.at[pl.ds(t * _GCH, _GCH)]],
                                    buf, sem)

        def wait(buf, sem):
            pltpu.make_async_copy(h_hbm.at[pl.ds(0, _GCH)], buf, sem).wait()

        def flush(t, buf):
            pltpu.sync_copy(buf, out_hbm.at[pl.ds(base + t * _GCH, _GCH)])

        issue(0, rows0, sem0)
        issue(1, rows1, sem1)

        def pair(p, carry):
            t0 = 2 * p
            wait(rows0, sem0)
            flush(t0, rows0)
            issue(t0 + 2, rows0, sem0)
            wait(rows1, sem1)
            flush(t0 + 1, rows1)
            issue(t0 + 3, rows1, sem1)
            return carry

        lax.fori_loop(0, npair, pair, 0)
        for t in range(2 * npair, nfull):
            b, s = bufs[t % 2], sems[t % 2]
            wait(b, s)
            flush(t, b)
            if t + 2 < nfull:
                issue(t + 2, b, s)
        if tail:
            b, s = bufs[nfull % 2], sems[nfull % 2]
            pltpu.async_copy(
                h_hbm.at[idx_v.at[pl.ds(nfull * _GCH, tail)]],
                b.at[pl.ds(0, tail)], s).wait()
            pltpu.sync_copy(b.at[pl.ds(0, tail)],
                            out_hbm.at[pl.ds(base + nfull * _GCH, tail)])

    return body


def _gather_sc(h, snd, n_e):
    mesh = plsc.VectorSubcoreMesh(core_axis_name="c", subcore_axis_name="s",
                                  num_cores=_NC, num_subcores=_NS)
    f = functools.partial(
        pl.kernel,
        out_type=jax.ShapeDtypeStruct((n_e, _DP), jnp.float32),
        mesh=mesh,
        scratch_types=[
            pltpu.VMEM((n_e // _NW,), jnp.int32),
            pltpu.VMEM((_GCH, _DP), jnp.float32),
            pltpu.VMEM((_GCH, _DP), jnp.float32),
            pltpu.SemaphoreType.DMA,
            pltpu.SemaphoreType.DMA,
        ],
    )(_make_gather_body(n_e // _NW))
    return f(h, snd)


def _edge_body(vec_ref, msgs_ref, wtp_ref, p_ref, w1_ref, w2_ref, w3_ref,
               out_ref):
    vec = vec_ref[...]
    xr, yr, zr = vec[0:1, :], vec[1:2, :], vec[2:3, :]
    l2 = xr * xr + yr * yr + zr * zr
    length = jnp.sqrt(l2)
    inv = 1.0 / length
    ns = lax.broadcasted_iota(jnp.int32, (8, 1), 0).astype(jnp.float32) + 1.0
    t = length * (1.0 / CUTOFF)
    t6 = t * t * t
    t6 = t6 * t6
    env = 1.0 - 28.0 * t6 + 48.0 * t6 * t - 21.0 * t6 * t * t
    env = jnp.where(t < 1.0, env, 0.0)
    radial_t = (jnp.sin(ns * (np.pi / CUTOFF) * length)
                * (np.float32(np.sqrt(2.0 / CUTOFF)) * inv * env))
    x = xr * inv
    y = yr * inv
    z = zr * inv
    x2, y2, z2 = x * x, y * y, z * z
    s3, s5, s7 = np.sqrt(3.0), np.sqrt(5.0), np.sqrt(7.0)
    s15, s105, s42, s70, s358 = (np.sqrt(15.0), np.sqrt(105.0),
                                 np.sqrt(42.0), np.sqrt(70.0),
                                 np.sqrt(35.0 / 8.0))
    sh_t = jnp.concatenate([
        s3 * x, s3 * y, s3 * z,
        s15 * x * y, s15 * y * z, (s5 / 2.0) * (3.0 * z2 - 1.0),
        s15 * x * z, (s15 / 2.0) * (x2 - y2),
        s358 * y * (3.0 * x2 - y2), s105 * x * y * z,
        (s42 / 4.0) * y * (5.0 * z2 - 1.0),
        (s7 / 2.0) * z * (5.0 * z2 - 3.0),
        (s42 / 4.0) * x * (5.0 * z2 - 1.0),
        (s105 / 2.0) * z * (x2 - y2),
        (s70 / 4.0) * x * (x2 - 3.0 * y2),
    ], axis=0)
    b1 = lax.dot_general(sh_t.astype(jnp.bfloat16), wtp_ref[...],
                         (((0,), (0,)), ((), ())),
                         preferred_element_type=jnp.float32)
    mix = _swish(lax.dot_general(radial_t, w1_ref[...],
                                 (((0,), (0,)), ((), ())),
                                 preferred_element_type=jnp.float32))
    mix = _swish(jnp.dot(mix, w2_ref[...],
                         preferred_element_type=jnp.float32))
    mix = jnp.dot(mix.astype(jnp.bfloat16), w3_ref[...],
                  preferred_element_type=jnp.float32)
    msgs = msgs_ref[...][:, :D]
    msgs_bf = msgs.astype(jnp.bfloat16)
    p = p_ref[...]
    tp = jnp.dot(msgs_bf[:, 0:NF], p,
                 preferred_element_type=jnp.float32) * b1[:, 0:D]
    for i in range(1, IR):
        tp = tp + jnp.dot(msgs_bf[:, NF * i:NF * (i + 1)], p,
                          preferred_element_type=jnp.float32) \
            * b1[:, _IRP * i:_IRP * i + D]
    be = msgs.shape[0]
    out_ref[...] = jnp.concatenate(
        [jnp.concatenate([msgs, tp], axis=1) * mix,
         jnp.zeros((be, _MP - MSG_DIM), jnp.float32)], axis=1)


def _edge_tc(vec_t, msgs, wtpq, ptile, w1, w2, w3p, n_e):
    be = 640
    return pl.pallas_call(
        _edge_body,
        grid=(n_e // be,),
        in_specs=[
            pl.BlockSpec((3, be), lambda i: (0, i)),
            pl.BlockSpec((be, _DP), lambda i: (i, 0)),
            pl.BlockSpec((SH_DIM, IR * _IRP), lambda i: (0, 0)),
            pl.BlockSpec((NF, D), lambda i: (0, 0)),
            pl.BlockSpec((8, 64), lambda i: (0, 0)),
            pl.BlockSpec((64, 64), lambda i: (0, 0)),
            pl.BlockSpec((64, MSG_DIM), lambda i: (0, 0)),
        ],
        out_specs=pl.BlockSpec((be, _MP), lambda i: (i, 0)),
        out_shape=jax.ShapeDtypeStruct((n_e, _MP), jnp.float32),
    )(vec_t, msgs, wtpq, ptile, w1, w2, w3p)


def _make_scatter_body(eps):
    nfull = eps // _SCH
    npair = (nfull - 2) // 2

    def body(msg_hbm, recv_hbm, z_hbm, agg_hbm,
             idx_v, buf0, buf1, stage, acc_sp, sem0, sem1):
        cid = lax.axis_index("c")
        sid = lax.axis_index("s")
        row0 = sid * _WBR
        nch = jnp.where(sid == _NS - 1, (N_NODES - (_NS - 1) * _WBR) // _WCH,
                        _WBR // _WCH)
        pltpu.sync_copy(recv_hbm.at[sid], idx_v)
        bufs, sems = (buf0, buf1), (sem0, sem1)

        for g_local in range(3):
            col0 = jnp.minimum(cid * 3 + g_local, _NGRP - 1) * _GRP
            pltpu.sync_copy(z_hbm, stage)

            def zero(k, carry):
                pltpu.sync_copy(stage,
                                acc_sp.at[pl.ds(row0 + k * _WCH, _WCH)])
                return carry

            lax.fori_loop(0, nch, zero, 0)
            plsc.subcore_barrier()

            def load(tc, buf, sem):
                return pltpu.async_copy(
                    msg_hbm.at[pl.ds(sid * eps + tc * _SCH, _SCH),
                               pl.ds(col0, _GRP)], buf, sem)

            def wait(buf, sem):
                pltpu.make_async_copy(
                    msg_hbm.at[pl.ds(0, _SCH), pl.ds(0, _GRP)],
                    buf, sem).wait()

            def scat(tc, buf):
                pltpu.sync_copy(buf, acc_sp.at[idx_v.at[tc]], add=True)

            load(0, buf0, sem0)
            load(1, buf1, sem1)

            def pair(p, carry):
                t0 = 2 * p
                wait(buf0, sem0)
                scat(t0, buf0)
                load(t0 + 2, buf0, sem0)
                wait(buf1, sem1)
                scat(t0 + 1, buf1)
                load(t0 + 3, buf1, sem1)
                return carry

            lax.fori_loop(0, npair, pair, 0)
            for t in range(2 * npair, nfull):
                b, s = bufs[t % 2], sems[t % 2]
                wait(b, s)
                scat(t, b)
                if t + 2 < nfull:
                    load(t + 2, b, s)
            plsc.subcore_barrier()

            def wb(k, carry):
                pltpu.sync_copy(acc_sp.at[pl.ds(row0 + k * _WCH, _WCH)],
                                stage)
                pltpu.sync_copy(stage,
                                agg_hbm.at[pl.ds(row0 + k * _WCH, _WCH),
                                           pl.ds(col0, _GRP)])
                return carry

            lax.fori_loop(0, nch, wb, 0)
            plsc.subcore_barrier()

    return body


def _scatter_sc(messages, rcv, zeros_grp, n_e):
    mesh = plsc.VectorSubcoreMesh(core_axis_name="c", subcore_axis_name="s",
                                  num_cores=_NC, num_subcores=_NS)
    eps = n_e // _NS
    f = functools.partial(
        pl.kernel,
        out_type=jax.ShapeDtypeStruct((N_NODES, _MP), jnp.float32),
        mesh=mesh,
        scratch_types=[
            pltpu.VMEM((eps // _SCH, _SCH), jnp.int32),
            pltpu.VMEM((_SCH, _GRP), jnp.float32),
            pltpu.VMEM((_SCH, _GRP), jnp.float32),
            pltpu.VMEM((_WCH, _GRP), jnp.float32),
            pltpu.VMEM_SHARED((N_NODES, _GRP), jnp.float32),
            pltpu.SemaphoreType.DMA,
            pltpu.SemaphoreType.DMA,
        ],
    )(_make_scatter_body(eps))
    return f(messages, rcv, zeros_grp)


def _final_body(agg_ref, agg2_ref, sc_ref, wd_ref, r1_ref, r2_ref, out_ref):
    xg = jnp.dot(agg_ref[...][:, :MSG_DIM] + agg2_ref[...][:, :MSG_DIM],
                 wd_ref[...], preferred_element_type=jnp.float32)
    s = xg[:, :NF]
    g1 = _swish(xg[:, NF:2 * NF])
    g2 = _swish(xg[:, 2 * NF:3 * NF])
    v1 = xg[:, 3 * NF:6 * NF]
    v2 = xg[:, 6 * NF:]
    s1 = jnp.dot(g1, r1_ref[...], preferred_element_type=jnp.float32)
    s2 = jnp.dot(g2, r2_ref[...], preferred_element_type=jnp.float32)
    out_ref[...] = jnp.concatenate([_swish(s), v1 * s1, v2 * s2],
                                   axis=1) + sc_ref[...]


def _final_tc(agg, agg2, sc, wd, r1, r2):
    bn = 2000
    return pl.pallas_call(
        _final_body,
        grid=(N_NODES // bn,),
        in_specs=[
            pl.BlockSpec((bn, _MP), lambda i: (i, 0)),
            pl.BlockSpec((bn, _MP), lambda i: (i, 0)),
            pl.BlockSpec((bn, D), lambda i: (i, 0)),
            pl.BlockSpec((MSG_DIM, GATE_DIM), lambda i: (0, 0)),
            pl.BlockSpec((NF, 3 * NF), lambda i: (0, 0)),
            pl.BlockSpec((NF, 5 * NF), lambda i: (0, 0)),
        ],
        out_specs=pl.BlockSpec((bn, D), lambda i: (i, 0)),
        out_shape=jax.ShapeDtypeStruct((N_NODES, D), jnp.float32),
    )(agg, agg2, sc, wd, r1, r2)


def kernel(vectors, node_feats, node_specie, senders, receivers,
           W_sc, W_up, W_tp, W1, W2, W3, W_down):
    wup_p = jnp.pad(W_up[:, _SIGMA], ((0, 0), (0, _DP - D)))
    wtp_r = jnp.transpose(W_tp, (1, 0, 2)).reshape(SH_DIM, IR * IR)
    src = np.concatenate([np.clip(np.arange(_IRP) // NF, 0, IR - 1) + IR * i
                          for i in range(IR)])
    msk = np.concatenate([(np.arange(_IRP) < D).astype(np.float32)] * IR)
    wtpq = (wtp_r[:, src] * msk[None, :]).astype(jnp.bfloat16)
    ptile = jnp.asarray(_P_TILE).astype(jnp.bfloat16)
    w3_p = W3[:, _PI_FULL].astype(jnp.bfloat16)
    wd_p = W_down[_PI_FULL, :] * np.float32(0.25)
    r1 = jnp.asarray(_R1)
    r2 = jnp.asarray(_R2)
    spec2d = node_specie.astype(jnp.int32).reshape(N_NODES, 1)
    snd = senders.astype(jnp.int32)
    rcv = receivers.astype(jnp.int32)
    zeros_grp = jnp.zeros((_WCH, _GRP), jnp.float32)
    vec_t = jnp.transpose(vectors)

    ea = _ESPL
    eb = N_EDGES - _ESPL
    h, sc = _node_pre(node_feats, spec2d, wup_p, W_sc)
    msgs_a = _gather_sc(h, snd[:ea], ea)
    msgs_b = _gather_sc(h, snd[ea:], eb)
    m_a = _edge_tc(vec_t[:, :ea], msgs_a, wtpq, ptile, W1, W2, w3_p, ea)
    m_b = _edge_tc(vec_t[:, ea:], msgs_b, wtpq, ptile, W1, W2, w3_p, eb)
    agg_a = _scatter_sc(m_a, rcv[:ea].reshape(_NS, ea // _NS // _SCH, _SCH),
                        zeros_grp, ea)
    agg_b = _scatter_sc(m_b, rcv[ea:].reshape(_NS, eb // _NS // _SCH, _SCH),
                        zeros_grp, eb)
    return _final_tc(agg_a, agg_b, sc, wd_p, r1, r2)


# speedup vs baseline: 5.3414x; 1.0681x over previous
import functools

import numpy as np
import jax
import jax.numpy as jnp
from jax import lax
from jax.experimental import pallas as pl
from jax.experimental.pallas import tpu as pltpu
from jax.experimental.pallas import tpu_sc as plsc

N_NODES = 10000
N_EDGES = 160000
NF = 32
IR = 9
D = NF * IR
SH_DIM = 15
MSG_DIM = 2 * D
GATE_DIM = D + 2 * NF
CUTOFF = 3.0

_SIGMA = np.arange(D).reshape(NF, IR).T.reshape(-1)
_PI_FULL = np.concatenate([_SIGMA, D + _SIGMA])

_P_TILE = np.tile(np.eye(NF, dtype=np.float32), (1, IR))
_IRP = 384

_R1 = np.zeros((NF, 3 * NF), np.float32)
_R1[np.repeat(np.arange(NF), 3), np.arange(3 * NF)] = 1.0
_R2 = np.zeros((NF, 5 * NF), np.float32)
_R2[np.repeat(np.arange(NF), 5), np.arange(5 * NF)] = 1.0

_DP = 384
_MP = 640
_NC, _NS = 2, 16
_NW = _NC * _NS
_GPW = N_EDGES // _NW
_GCH = 128
_GFULL = _GPW // _GCH
_GTAIL = _GPW - _GFULL * _GCH
_SPS = N_EDGES // _NS
_SCH = 80
_SNCH = _SPS // _SCH
_GRP = 128
_PARTS = (64000, 51200, 44800)
_NGRP = _MP // _GRP
_WBR = 640
_WCH = 80


def _swish(x):
    return x / (1.0 + jnp.exp(-x))


def _h_body(x_ref, wup_ref, h_ref):
    h_ref[...] = jnp.dot(x_ref[...], wup_ref[...],
                         preferred_element_type=jnp.float32)


def _h_tc(x, wup):
    bn = 2000
    return pl.pallas_call(
        _h_body,
        grid=(N_NODES // bn,),
        in_specs=[
            pl.BlockSpec((bn, D), lambda i: (i, 0)),
            pl.BlockSpec((D, _DP), lambda i: (0, 0)),
        ],
        out_specs=pl.BlockSpec((bn, _DP), lambda i: (i, 0)),
        out_shape=jax.ShapeDtypeStruct((N_NODES, _DP), jnp.float32),
    )(x, wup)


def _sc_body(x_ref, spec_ref, wsc_ref, sc_ref):
    x = x_ref[...]
    spec = spec_ref[...]
    acc = (spec == 0).astype(jnp.float32) * jnp.dot(
        x, wsc_ref[0], preferred_element_type=jnp.float32)
    for s in range(1, 5):
        acc = acc + (spec == s).astype(jnp.float32) * jnp.dot(
            x, wsc_ref[s], preferred_element_type=jnp.float32)
    sc_ref[...] = acc


def _sc_tc(x, spec2d, wsc):
    bn = 2000
    return pl.pallas_call(
        _sc_body,
        grid=(N_NODES // bn,),
        in_specs=[
            pl.BlockSpec((bn, D), lambda i: (i, 0)),
            pl.BlockSpec((bn, 1), lambda i: (i, 0)),
            pl.BlockSpec((5, D, D), lambda i: (0, 0, 0)),
        ],
        out_specs=pl.BlockSpec((bn, D), lambda i: (i, 0)),
        out_shape=jax.ShapeDtypeStruct((N_NODES, D), jnp.float32),
    )(x, spec2d, wsc)


def _make_gather_body(rows_pw):
    nfull = rows_pw // _GCH
    tail = rows_pw - nfull * _GCH
    npair = (nfull - 2) // 2

    def body(h_hbm, snd_hbm, out_hbm, idx_v, rows0, rows1, sem0, sem1):
        wid = lax.axis_index("s") * _NC + lax.axis_index("c")
        base = wid * rows_pw
        pltpu.sync_copy(snd_hbm.at[pl.ds(base, rows_pw)], idx_v)
        bufs, sems = (rows0, rows1), (sem0, sem1)

        def issue(t, buf, sem):
            return pltpu.async_copy(h_hbm.at[idx_v.at[pl.ds(t * _GCH, _GCH)]],
                                    buf, sem)

        def wait(buf, sem):
            pltpu.make_async_copy(h_hbm.at[pl.ds(0, _GCH)], buf, sem).wait()

        def flush(t, buf):
            pltpu.sync_copy(buf, out_hbm.at[pl.ds(base + t * _GCH, _GCH)])

        issue(0, rows0, sem0)
        issue(1, rows1, sem1)

        def pair(p, carry):
            t0 = 2 * p
            wait(rows0, sem0)
            flush(t0, rows0)
            issue(t0 + 2, rows0, sem0)
            wait(rows1, sem1)
            flush(t0 + 1, rows1)
            issue(t0 + 3, rows1, sem1)
            return carry

        lax.fori_loop(0, npair, pair, 0)
        for t in range(2 * npair, nfull):
            b, s = bufs[t % 2], sems[t % 2]
            wait(b, s)
            flush(t, b)
            if t + 2 < nfull:
                issue(t + 2, b, s)
        if tail:
            b, s = bufs[nfull % 2], sems[nfull % 2]
            pltpu.async_copy(
                h_hbm.at[idx_v.at[pl.ds(nfull * _GCH, tail)]],
                b.at[pl.ds(0, tail)], s).wait()
            pltpu.sync_copy(b.at[pl.ds(0, tail)],
                            out_hbm.at[pl.ds(base + nfull * _GCH, tail)])

    return body


def _gather_sc(h, snd, n_e):
    mesh = plsc.VectorSubcoreMesh(core_axis_name="c", subcore_axis_name="s",
                                  num_cores=_NC, num_subcores=_NS)
    f = functools.partial(
        pl.kernel,
        out_type=jax.ShapeDtypeStruct((n_e, _DP), jnp.float32),
        mesh=mesh,
        scratch_types=[
            pltpu.VMEM((n_e // _NW,), jnp.int32),
            pltpu.VMEM((_GCH, _DP), jnp.float32),
            pltpu.VMEM((_GCH, _DP), jnp.float32),
            pltpu.SemaphoreType.DMA,
            pltpu.SemaphoreType.DMA,
        ],
    )(_make_gather_body(n_e // _NW))
    return f(h, snd)


def _edge_body(vec_ref, msgs_ref, wtp_ref, p_ref, w1_ref, w2_ref, w3_ref,
               out_ref):
    vec = vec_ref[...]
    xr, yr, zr = vec[0:1, :], vec[1:2, :], vec[2:3, :]
    l2 = xr * xr + yr * yr + zr * zr
    length = jnp.sqrt(l2)
    inv = 1.0 / length
    ns = lax.broadcasted_iota(jnp.int32, (8, 1), 0).astype(jnp.float32) + 1.0
    t = length * (1.0 / CUTOFF)
    t6 = t * t * t
    t6 = t6 * t6
    env = 1.0 - 28.0 * t6 + 48.0 * t6 * t - 21.0 * t6 * t * t
    env = jnp.where(t < 1.0, env, 0.0)
    radial_t = (jnp.sin(ns * (np.pi / CUTOFF) * length)
                * (np.float32(np.sqrt(2.0 / CUTOFF)) * inv * env))
    x = xr * inv
    y = yr * inv
    z = zr * inv
    x2, y2, z2 = x * x, y * y, z * z
    s3, s5, s7 = np.sqrt(3.0), np.sqrt(5.0), np.sqrt(7.0)
    s15, s105, s42, s70, s358 = (np.sqrt(15.0), np.sqrt(105.0),
                                 np.sqrt(42.0), np.sqrt(70.0),
                                 np.sqrt(35.0 / 8.0))
    sh_t = jnp.concatenate([
        s3 * x, s3 * y, s3 * z,
        s15 * x * y, s15 * y * z, (s5 / 2.0) * (3.0 * z2 - 1.0),
        s15 * x * z, (s15 / 2.0) * (x2 - y2),
        s358 * y * (3.0 * x2 - y2), s105 * x * y * z,
        (s42 / 4.0) * y * (5.0 * z2 - 1.0),
        (s7 / 2.0) * z * (5.0 * z2 - 3.0),
        (s42 / 4.0) * x * (5.0 * z2 - 1.0),
        (s105 / 2.0) * z * (x2 - y2),
        (s70 / 4.0) * x * (x2 - 3.0 * y2),
    ], axis=0)
    b1 = lax.dot_general(sh_t.astype(jnp.bfloat16), wtp_ref[...],
                         (((0,), (0,)), ((), ())),
                         preferred_element_type=jnp.float32)
    mix = _swish(lax.dot_general(radial_t, w1_ref[...],
                                 (((0,), (0,)), ((), ())),
                                 preferred_element_type=jnp.float32))
    mix = _swish(jnp.dot(mix, w2_ref[...],
                         preferred_element_type=jnp.float32))
    mix = jnp.dot(mix.astype(jnp.bfloat16), w3_ref[...],
                  preferred_element_type=jnp.float32)
    msgs = msgs_ref[...][:, :D]
    msgs_bf = msgs.astype(jnp.bfloat16)
    p = p_ref[...]
    tp = jnp.dot(msgs_bf[:, 0:NF], p,
                 preferred_element_type=jnp.float32) * b1[:, 0:D]
    for i in range(1, IR):
        tp = tp + jnp.dot(msgs_bf[:, NF * i:NF * (i + 1)], p,
                          preferred_element_type=jnp.float32) \
            * b1[:, _IRP * i:_IRP * i + D]
    be = msgs.shape[0]
    out_ref[...] = jnp.concatenate(
        [jnp.concatenate([msgs, tp], axis=1) * mix,
         jnp.zeros((be, _MP - MSG_DIM), jnp.float32)], axis=1)


def _edge_tc(vec_t, msgs, wtpq, ptile, w1, w2, w3p, n_e):
    be = 640
    return pl.pallas_call(
        _edge_body,
        grid=(n_e // be,),
        in_specs=[
            pl.BlockSpec((3, be), lambda i: (0, i)),
            pl.BlockSpec((be, _DP), lambda i: (i, 0)),
            pl.BlockSpec((SH_DIM, IR * _IRP), lambda i: (0, 0)),
            pl.BlockSpec((NF, D), lambda i: (0, 0)),
            pl.BlockSpec((8, 64), lambda i: (0, 0)),
            pl.BlockSpec((64, 64), lambda i: (0, 0)),
            pl.BlockSpec((64, MSG_DIM), lambda i: (0, 0)),
        ],
        out_specs=pl.BlockSpec((be, _MP), lambda i: (i, 0)),
        out_shape=jax.ShapeDtypeStruct((n_e, _MP), jnp.float32),
    )(vec_t, msgs, wtpq, ptile, w1, w2, w3p)


def _make_scatter_body(eps):
    nfull = eps // _SCH
    npair = (nfull - 2) // 2

    def body(msg_hbm, recv_hbm, z_hbm, agg_hbm,
             idx_v, buf0, buf1, stage, acc_sp, sem0, sem1):
        cid = lax.axis_index("c")
        sid = lax.axis_index("s")
        row0 = sid * _WBR
        nch = jnp.where(sid == _NS - 1, (N_NODES - (_NS - 1) * _WBR) // _WCH,
                        _WBR // _WCH)
        pltpu.sync_copy(recv_hbm.at[sid], idx_v)
        bufs, sems = (buf0, buf1), (sem0, sem1)

        for g_local in range(3):
            col0 = jnp.minimum(cid * 3 + g_local, _NGRP - 1) * _GRP
            pltpu.sync_copy(z_hbm, stage)

            def zero(k, carry):
                pltpu.sync_copy(stage,
                                acc_sp.at[pl.ds(row0 + k * _WCH, _WCH)])
                return carry

            lax.fori_loop(0, nch, zero, 0)
            plsc.subcore_barrier()

            def load(tc, buf, sem):
                return pltpu.async_copy(
                    msg_hbm.at[pl.ds(sid * eps + tc * _SCH, _SCH),
                               pl.ds(col0, _GRP)], buf, sem)

            def wait(buf, sem):
                pltpu.make_async_copy(
                    msg_hbm.at[pl.ds(0, _SCH), pl.ds(0, _GRP)],
                    buf, sem).wait()

            def scat(tc, buf):
                pltpu.sync_copy(buf, acc_sp.at[idx_v.at[tc]], add=True)

            load(0, buf0, sem0)
            load(1, buf1, sem1)

            def pair(p, carry):
                t0 = 2 * p
                wait(buf0, sem0)
                scat(t0, buf0)
                load(t0 + 2, buf0, sem0)
                wait(buf1, sem1)
                scat(t0 + 1, buf1)
                load(t0 + 3, buf1, sem1)
                return carry

            lax.fori_loop(0, npair, pair, 0)
            for t in range(2 * npair, nfull):
                b, s = bufs[t % 2], sems[t % 2]
                wait(b, s)
                scat(t, b)
                if t + 2 < nfull:
                    load(t + 2, b, s)
            plsc.subcore_barrier()

            def wb(k, carry):
                pltpu.sync_copy(acc_sp.at[pl.ds(row0 + k * _WCH, _WCH)],
                                stage)
                pltpu.sync_copy(stage,
                                agg_hbm.at[pl.ds(row0 + k * _WCH, _WCH),
                                           pl.ds(col0, _GRP)])
                return carry

            lax.fori_loop(0, nch, wb, 0)
            plsc.subcore_barrier()

    return body


def _scatter_sc(messages, rcv, zeros_grp, n_e):
    mesh = plsc.VectorSubcoreMesh(core_axis_name="c", subcore_axis_name="s",
                                  num_cores=_NC, num_subcores=_NS)
    eps = n_e // _NS
    f = functools.partial(
        pl.kernel,
        out_type=jax.ShapeDtypeStruct((N_NODES, _MP), jnp.float32),
        mesh=mesh,
        scratch_types=[
            pltpu.VMEM((eps // _SCH, _SCH), jnp.int32),
            pltpu.VMEM((_SCH, _GRP), jnp.float32),
            pltpu.VMEM((_SCH, _GRP), jnp.float32),
            pltpu.VMEM((_WCH, _GRP), jnp.float32),
            pltpu.VMEM_SHARED((N_NODES, _GRP), jnp.float32),
            pltpu.SemaphoreType.DMA,
            pltpu.SemaphoreType.DMA,
        ],
    )(_make_scatter_body(eps))
    return f(messages, rcv, zeros_grp)


def _final_body(agg_ref, agg2_ref, agg3_ref, sc_ref, wd_ref, r1_ref, r2_ref,
                out_ref):
    xg = jnp.dot(agg_ref[...][:, :MSG_DIM] + agg2_ref[...][:, :MSG_DIM]
                 + agg3_ref[...][:, :MSG_DIM],
                 wd_ref[...], preferred_element_type=jnp.float32)
    s = xg[:, :NF]
    g1 = _swish(xg[:, NF:2 * NF])
    g2 = _swish(xg[:, 2 * NF:3 * NF])
    v1 = xg[:, 3 * NF:6 * NF]
    v2 = xg[:, 6 * NF:]
    s1 = jnp.dot(g1, r1_ref[...], preferred_element_type=jnp.float32)
    s2 = jnp.dot(g2, r2_ref[...], preferred_element_type=jnp.float32)
    out_ref[...] = jnp.concatenate([_swish(s), v1 * s1, v2 * s2],
                                   axis=1) + sc_ref[...]


def _final_tc(agg, agg2, agg3, sc, wd, r1, r2):
    bn = 2000
    return pl.pallas_call(
        _final_body,
        grid=(N_NODES // bn,),
        in_specs=[
            pl.BlockSpec((bn, _MP), lambda i: (i, 0)),
            pl.BlockSpec((bn, _MP), lambda i: (i, 0)),
            pl.BlockSpec((bn, _MP), lambda i: (i, 0)),
            pl.BlockSpec((bn, D), lambda i: (i, 0)),
            pl.BlockSpec((MSG_DIM, GATE_DIM), lambda i: (0, 0)),
            pl.BlockSpec((NF, 3 * NF), lambda i: (0, 0)),
            pl.BlockSpec((NF, 5 * NF), lambda i: (0, 0)),
        ],
        out_specs=pl.BlockSpec((bn, D), lambda i: (i, 0)),
        out_shape=jax.ShapeDtypeStruct((N_NODES, D), jnp.float32),
    )(agg, agg2, agg3, sc, wd, r1, r2)


def kernel(vectors, node_feats, node_specie, senders, receivers,
           W_sc, W_up, W_tp, W1, W2, W3, W_down):
    wup_p = jnp.pad(W_up[:, _SIGMA], ((0, 0), (0, _DP - D)))
    wtp_r = jnp.transpose(W_tp, (1, 0, 2)).reshape(SH_DIM, IR * IR)
    src = np.concatenate([np.clip(np.arange(_IRP) // NF, 0, IR - 1) + IR * i
                          for i in range(IR)])
    msk = np.concatenate([(np.arange(_IRP) < D).astype(np.float32)] * IR)
    wtpq = (wtp_r[:, src] * msk[None, :]).astype(jnp.bfloat16)
    ptile = jnp.asarray(_P_TILE).astype(jnp.bfloat16)
    w3_p = W3[:, _PI_FULL].astype(jnp.bfloat16)
    wd_p = W_down[_PI_FULL, :] * np.float32(0.25)
    r1 = jnp.asarray(_R1)
    r2 = jnp.asarray(_R2)
    spec2d = node_specie.astype(jnp.int32).reshape(N_NODES, 1)
    snd = senders.astype(jnp.int32)
    rcv = receivers.astype(jnp.int32)
    zeros_grp = jnp.zeros((_WCH, _GRP), jnp.float32)
    vec_t = jnp.transpose(vectors)

    h = _h_tc(node_feats, wup_p)
    sc = _sc_tc(node_feats, spec2d, W_sc)
    aggs = []
    off = 0
    for ne in _PARTS:
        sl = slice(off, off + ne)
        msgs = _gather_sc(h, snd[sl], ne)
        m = _edge_tc(vec_t[:, sl], msgs, wtpq, ptile, W1, W2, w3_p, ne)
        aggs.append(_scatter_sc(
            m, rcv[sl].reshape(_NS, ne // _NS // _SCH, _SCH),
            zeros_grp, ne))
        off += ne
    return _final_tc(aggs[0], aggs[1], aggs[2], sc, wd_p, r1, r2)


# speedup vs baseline: 5.6651x; 1.0606x over previous
import functools

import numpy as np
import jax
import jax.numpy as jnp
from jax import lax
from jax.experimental import pallas as pl
from jax.experimental.pallas import tpu as pltpu
from jax.experimental.pallas import tpu_sc as plsc

N_NODES = 10000
N_EDGES = 160000
NF = 32
IR = 9
D = NF * IR
SH_DIM = 15
MSG_DIM = 2 * D
GATE_DIM = D + 2 * NF
CUTOFF = 3.0

_SIGMA = np.arange(D).reshape(NF, IR).T.reshape(-1)
_PI_FULL = np.concatenate([_SIGMA, D + _SIGMA])

_P_TILE = np.tile(np.eye(NF, dtype=np.float32), (1, IR))
_IRP = 384

_R1 = np.zeros((NF, 3 * NF), np.float32)
_R1[np.repeat(np.arange(NF), 3), np.arange(3 * NF)] = 1.0
_R2 = np.zeros((NF, 5 * NF), np.float32)
_R2[np.repeat(np.arange(NF), 5), np.arange(5 * NF)] = 1.0

_DP = 384
_MP = 640
_NC, _NS = 2, 16
_NW = _NC * _NS
_GPW = N_EDGES // _NW
_GCH = 128
_GFULL = _GPW // _GCH
_GTAIL = _GPW - _GFULL * _GCH
_SPS = N_EDGES // _NS
_SCH = 80
_SNCH = _SPS // _SCH
_GRP = 128
_PARTS = (64000, 51200, 44800)
_NGRP = _MP // _GRP
_WBR = 640
_WCH = 80


def _swish(x):
    return x / (1.0 + jnp.exp(-x))


def _h_body(x_ref, wup_ref, h_ref):
    h_ref[...] = jnp.dot(x_ref[...], wup_ref[...],
                         preferred_element_type=jnp.float32)


def _h_tc(x, wup):
    bn = 2000
    return pl.pallas_call(
        _h_body,
        grid=(N_NODES // bn,),
        in_specs=[
            pl.BlockSpec((bn, D), lambda i: (i, 0)),
            pl.BlockSpec((D, _DP), lambda i: (0, 0)),
        ],
        out_specs=pl.BlockSpec((bn, _DP), lambda i: (i, 0)),
        out_shape=jax.ShapeDtypeStruct((N_NODES, _DP), jnp.float32),
    )(x, wup)


def _sc_body(x_ref, spec_ref, wsc_ref, sc_ref):
    x = x_ref[...]
    spec = spec_ref[...]
    acc = (spec == 0).astype(jnp.float32) * jnp.dot(
        x, wsc_ref[0], preferred_element_type=jnp.float32)
    for s in range(1, 5):
        acc = acc + (spec == s).astype(jnp.float32) * jnp.dot(
            x, wsc_ref[s], preferred_element_type=jnp.float32)
    sc_ref[...] = acc


def _sc_tc(x, spec2d, wsc):
    bn = 2000
    return pl.pallas_call(
        _sc_body,
        grid=(N_NODES // bn,),
        in_specs=[
            pl.BlockSpec((bn, D), lambda i: (i, 0)),
            pl.BlockSpec((bn, 1), lambda i: (i, 0)),
            pl.BlockSpec((5, D, D), lambda i: (0, 0, 0)),
        ],
        out_specs=pl.BlockSpec((bn, D), lambda i: (i, 0)),
        out_shape=jax.ShapeDtypeStruct((N_NODES, D), jnp.float32),
    )(x, spec2d, wsc)


def _make_gather_body(rows_pw):
    nfull = rows_pw // _GCH
    tail = rows_pw - nfull * _GCH
    npair = (nfull - 2) // 2

    def body(h_hbm, snd_hbm, out_hbm, idx_v, rows0, rows1, sem0, sem1):
        wid = lax.axis_index("s") * _NC + lax.axis_index("c")
        base = wid * rows_pw
        pltpu.sync_copy(snd_hbm.at[pl.ds(base, rows_pw)], idx_v)
        bufs, sems = (rows0, rows1), (sem0, sem1)

        def issue(t, buf, sem):
            return pltpu.async_copy(h_hbm.at[idx_v.at[pl.ds(t * _GCH, _GCH)]],
                                    buf, sem)

        def wait(buf, sem):
            pltpu.make_async_copy(h_hbm.at[pl.ds(0, _GCH)], buf, sem).wait()

        def flush(t, buf):
            pltpu.sync_copy(buf, out_hbm.at[pl.ds(base + t * _GCH, _GCH)])

        issue(0, rows0, sem0)
        issue(1, rows1, sem1)

        def pair(p, carry):
            t0 = 2 * p
            wait(rows0, sem0)
            flush(t0, rows0)
            issue(t0 + 2, rows0, sem0)
            wait(rows1, sem1)
            flush(t0 + 1, rows1)
            issue(t0 + 3, rows1, sem1)
            return carry

        lax.fori_loop(0, npair, pair, 0)
        for t in range(2 * npair, nfull):
            b, s = bufs[t % 2], sems[t % 2]
            wait(b, s)
            flush(t, b)
            if t + 2 < nfull:
                issue(t + 2, b, s)
        if tail:
            b, s = bufs[nfull % 2], sems[nfull % 2]
            pltpu.async_copy(
                h_hbm.at[idx_v.at[pl.ds(nfull * _GCH, tail)]],
                b.at[pl.ds(0, tail)], s).wait()
            pltpu.sync_copy(b.at[pl.ds(0, tail)],
                            out_hbm.at[pl.ds(base + nfull * _GCH, tail)])

    return body


def _gather_sc(h, snd, n_e):
    mesh = plsc.VectorSubcoreMesh(core_axis_name="c", subcore_axis_name="s",
                                  num_cores=_NC, num_subcores=_NS)
    f = functools.partial(
        pl.kernel,
        out_type=jax.ShapeDtypeStruct((n_e, _DP), jnp.float32),
        mesh=mesh,
        scratch_types=[
            pltpu.VMEM((n_e // _NW,), jnp.int32),
            pltpu.VMEM((_GCH, _DP), jnp.float32),
            pltpu.VMEM((_GCH, _DP), jnp.float32),
            pltpu.SemaphoreType.DMA,
            pltpu.SemaphoreType.DMA,
        ],
    )(_make_gather_body(n_e // _NW))
    return f(h, snd)


def _edge_body(vec_ref, msgs_ref, wtp_ref, p_ref, w1_ref, w2_ref, w3_ref,
               out_ref):
    vec = vec_ref[...]
    xr, yr, zr = vec[0:1, :], vec[1:2, :], vec[2:3, :]
    l2 = xr * xr + yr * yr + zr * zr
    length = jnp.sqrt(l2)
    inv = 1.0 / length
    ns = lax.broadcasted_iota(jnp.int32, (8, 1), 0).astype(jnp.float32) + 1.0
    t = length * (1.0 / CUTOFF)
    t6 = t * t * t
    t6 = t6 * t6
    env = 1.0 - 28.0 * t6 + 48.0 * t6 * t - 21.0 * t6 * t * t
    env = jnp.where(t < 1.0, env, 0.0)
    radial_t = (jnp.sin(ns * (np.pi / CUTOFF) * length)
                * (np.float32(np.sqrt(2.0 / CUTOFF)) * inv * env))
    x = xr * inv
    y = yr * inv
    z = zr * inv
    x2, y2, z2 = x * x, y * y, z * z
    s3, s5, s7 = np.sqrt(3.0), np.sqrt(5.0), np.sqrt(7.0)
    s15, s105, s42, s70, s358 = (np.sqrt(15.0), np.sqrt(105.0),
                                 np.sqrt(42.0), np.sqrt(70.0),
                                 np.sqrt(35.0 / 8.0))
    sh_t = jnp.concatenate([
        s3 * x, s3 * y, s3 * z,
        s15 * x * y, s15 * y * z, (s5 / 2.0) * (3.0 * z2 - 1.0),
        s15 * x * z, (s15 / 2.0) * (x2 - y2),
        s358 * y * (3.0 * x2 - y2), s105 * x * y * z,
        (s42 / 4.0) * y * (5.0 * z2 - 1.0),
        (s7 / 2.0) * z * (5.0 * z2 - 3.0),
        (s42 / 4.0) * x * (5.0 * z2 - 1.0),
        (s105 / 2.0) * z * (x2 - y2),
        (s70 / 4.0) * x * (x2 - 3.0 * y2),
    ], axis=0)
    b1 = lax.dot_general(sh_t.astype(jnp.bfloat16), wtp_ref[...],
                         (((0,), (0,)), ((), ())),
                         preferred_element_type=jnp.float32
                         ).astype(jnp.bfloat16)
    mix = _swish(lax.dot_general(radial_t, w1_ref[...],
                                 (((0,), (0,)), ((), ())),
                                 preferred_element_type=jnp.float32))
    mix = _swish(jnp.dot(mix, w2_ref[...],
                         preferred_element_type=jnp.float32))
    mix = jnp.dot(mix.astype(jnp.bfloat16), w3_ref[...],
                  preferred_element_type=jnp.float32)
    msgs = msgs_ref[...][:, :D]
    msgs_bf = msgs.astype(jnp.bfloat16)
    p = p_ref[...]
    tp = jnp.dot(msgs_bf[:, 0:NF], p,
                 preferred_element_type=jnp.float32) \
        * b1[:, 0:D].astype(jnp.float32)
    for i in range(1, IR):
        tp = tp + jnp.dot(msgs_bf[:, NF * i:NF * (i + 1)], p,
                          preferred_element_type=jnp.float32) \
            * b1[:, _IRP * i:_IRP * i + D].astype(jnp.float32)
    be = msgs.shape[0]
    out_ref[...] = jnp.concatenate(
        [jnp.concatenate([msgs, tp], axis=1) * mix,
         jnp.zeros((be, _MP - MSG_DIM), jnp.float32)], axis=1)


def _edge_tc(vec_t, msgs, wtpq, ptile, w1, w2, w3p, n_e):
    be = 1280
    return pl.pallas_call(
        _edge_body,
        grid=(n_e // be,),
        in_specs=[
            pl.BlockSpec((3, be), lambda i: (0, i)),
            pl.BlockSpec((be, _DP), lambda i: (i, 0)),
            pl.BlockSpec((SH_DIM, IR * _IRP), lambda i: (0, 0)),
            pl.BlockSpec((NF, D), lambda i: (0, 0)),
            pl.BlockSpec((8, 64), lambda i: (0, 0)),
            pl.BlockSpec((64, 64), lambda i: (0, 0)),
            pl.BlockSpec((64, MSG_DIM), lambda i: (0, 0)),
        ],
        out_specs=pl.BlockSpec((be, _MP), lambda i: (i, 0)),
        out_shape=jax.ShapeDtypeStruct((n_e, _MP), jnp.float32),
    )(vec_t, msgs, wtpq, ptile, w1, w2, w3p)


def _make_scatter_body(eps):
    nfull = eps // _SCH
    npair = (nfull - 2) // 2

    def body(msg_hbm, recv_hbm, z_hbm, agg_hbm,
             idx_v, buf0, buf1, stage, acc_sp, sem0, sem1):
        cid = lax.axis_index("c")
        sid = lax.axis_index("s")
        row0 = sid * _WBR
        nch = jnp.where(sid == _NS - 1, (N_NODES - (_NS - 1) * _WBR) // _WCH,
                        _WBR // _WCH)
        pltpu.sync_copy(recv_hbm.at[sid], idx_v)
        bufs, sems = (buf0, buf1), (sem0, sem1)

        for g_local in range(3):
            col0 = jnp.minimum(cid * 3 + g_local, _NGRP - 1) * _GRP
            pltpu.sync_copy(z_hbm, stage)

            def zero(k, carry):
                pltpu.sync_copy(stage,
                                acc_sp.at[pl.ds(row0 + k * _WCH, _WCH)])
                return carry

            lax.fori_loop(0, nch, zero, 0)
            plsc.subcore_barrier()

            def load(tc, buf, sem):
                return pltpu.async_copy(
                    msg_hbm.at[pl.ds(sid * eps + tc * _SCH, _SCH),
                               pl.ds(col0, _GRP)], buf, sem)

            def wait(buf, sem):
                pltpu.make_async_copy(
                    msg_hbm.at[pl.ds(0, _SCH), pl.ds(0, _GRP)],
                    buf, sem).wait()

            def scat(tc, buf):
                pltpu.sync_copy(buf, acc_sp.at[idx_v.at[tc]], add=True)

            load(0, buf0, sem0)
            load(1, buf1, sem1)

            def pair(p, carry):
                t0 = 2 * p
                wait(buf0, sem0)
                scat(t0, buf0)
                load(t0 + 2, buf0, sem0)
                wait(buf1, sem1)
                scat(t0 + 1, buf1)
                load(t0 + 3, buf1, sem1)
                return carry

            lax.fori_loop(0, npair, pair, 0)
            for t in range(2 * npair, nfull):
                b, s = bufs[t % 2], sems[t % 2]
                wait(b, s)
                scat(t, b)
                if t + 2 < nfull:
                    load(t + 2, b, s)
            plsc.subcore_barrier()

            def wb(k, carry):
                pltpu.sync_copy(acc_sp.at[pl.ds(row0 + k * _WCH, _WCH)],
                                stage)
                pltpu.sync_copy(stage,
                                agg_hbm.at[pl.ds(row0 + k * _WCH, _WCH),
                                           pl.ds(col0, _GRP)])
                return carry

            lax.fori_loop(0, nch, wb, 0)
            plsc.subcore_barrier()

    return body


def _scatter_sc(messages, rcv, zeros_grp, n_e):
    mesh = plsc.VectorSubcoreMesh(core_axis_name="c", subcore_axis_name="s",
                                  num_cores=_NC, num_subcores=_NS)
    eps = n_e // _NS
    f = functools.partial(
        pl.kernel,
        out_type=jax.ShapeDtypeStruct((N_NODES, _MP), jnp.float32),
        mesh=mesh,
        scratch_types=[
            pltpu.VMEM((eps // _SCH, _SCH), jnp.int32),
            pltpu.VMEM((_SCH, _GRP), jnp.float32),
            pltpu.VMEM((_SCH, _GRP), jnp.float32),
            pltpu.VMEM((_WCH, _GRP), jnp.float32),
            pltpu.VMEM_SHARED((N_NODES, _GRP), jnp.float32),
            pltpu.SemaphoreType.DMA,
            pltpu.SemaphoreType.DMA,
        ],
    )(_make_scatter_body(eps))
    return f(messages, rcv, zeros_grp)


def _final_body(agg_ref, agg2_ref, agg3_ref, sc_ref, wd_ref, r1_ref, r2_ref,
                out_ref):
    xg = jnp.dot(agg_ref[...][:, :MSG_DIM] + agg2_ref[...][:, :MSG_DIM]
                 + agg3_ref[...][:, :MSG_DIM],
                 wd_ref[...], preferred_element_type=jnp.float32)
    s = xg[:, :NF]
    g1 = _swish(xg[:, NF:2 * NF])
    g2 = _swish(xg[:, 2 * NF:3 * NF])
    v1 = xg[:, 3 * NF:6 * NF]
    v2 = xg[:, 6 * NF:]
    s1 = jnp.dot(g1, r1_ref[...], preferred_element_type=jnp.float32)
    s2 = jnp.dot(g2, r2_ref[...], preferred_element_type=jnp.float32)
    out_ref[...] = jnp.concatenate([_swish(s), v1 * s1, v2 * s2],
                                   axis=1) + sc_ref[...]


def _final_tc(agg, agg2, agg3, sc, wd, r1, r2):
    bn = 2000
    return pl.pallas_call(
        _final_body,
        grid=(N_NODES // bn,),
        in_specs=[
            pl.BlockSpec((bn, _MP), lambda i: (i, 0)),
            pl.BlockSpec((bn, _MP), lambda i: (i, 0)),
            pl.BlockSpec((bn, _MP), lambda i: (i, 0)),
            pl.BlockSpec((bn, D), lambda i: (i, 0)),
            pl.BlockSpec((MSG_DIM, GATE_DIM), lambda i: (0, 0)),
            pl.BlockSpec((NF, 3 * NF), lambda i: (0, 0)),
            pl.BlockSpec((NF, 5 * NF), lambda i: (0, 0)),
        ],
        out_specs=pl.BlockSpec((bn, D), lambda i: (i, 0)),
        out_shape=jax.ShapeDtypeStruct((N_NODES, D), jnp.float32),
    )(agg, agg2, agg3, sc, wd, r1, r2)


def kernel(vectors, node_feats, node_specie, senders, receivers,
           W_sc, W_up, W_tp, W1, W2, W3, W_down):
    wup_p = jnp.pad(W_up[:, _SIGMA], ((0, 0), (0, _DP - D)))
    wtp_r = jnp.transpose(W_tp, (1, 0, 2)).reshape(SH_DIM, IR * IR)
    src = np.concatenate([np.clip(np.arange(_IRP) // NF, 0, IR - 1) + IR * i
                          for i in range(IR)])
    msk = np.concatenate([(np.arange(_IRP) < D).astype(np.float32)] * IR)
    wtpq = (wtp_r[:, src] * msk[None, :]).astype(jnp.bfloat16)
    ptile = jnp.asarray(_P_TILE).astype(jnp.bfloat16)
    w3_p = W3[:, _PI_FULL].astype(jnp.bfloat16)
    wd_p = W_down[_PI_FULL, :] * np.float32(0.25)
    r1 = jnp.asarray(_R1)
    r2 = jnp.asarray(_R2)
    spec2d = node_specie.astype(jnp.int32).reshape(N_NODES, 1)
    snd = senders.astype(jnp.int32)
    rcv = receivers.astype(jnp.int32)
    zeros_grp = jnp.zeros((_WCH, _GRP), jnp.float32)
    vec_t = jnp.transpose(vectors)

    h = _h_tc(node_feats, wup_p)
    sc = _sc_tc(node_feats, spec2d, W_sc)
    aggs = []
    off = 0
    for ne in _PARTS:
        sl = slice(off, off + ne)
        msgs = _gather_sc(h, snd[sl], ne)
        m = _edge_tc(vec_t[:, sl], msgs, wtpq, ptile, W1, W2, w3_p, ne)
        aggs.append(_scatter_sc(
            m, rcv[sl].reshape(_NS, ne // _NS // _SCH, _SCH),
            zeros_grp, ne))
        off += ne
    return _final_tc(aggs[0], aggs[1], aggs[2], sc, wd_p, r1, r2)


# speedup vs baseline: 5.8007x; 1.0239x over previous
import functools

import numpy as np
import jax
import jax.numpy as jnp
from jax import lax
from jax.experimental import pallas as pl
from jax.experimental.pallas import tpu as pltpu
from jax.experimental.pallas import tpu_sc as plsc

N_NODES = 10000
N_EDGES = 160000
NF = 32
IR = 9
D = NF * IR
SH_DIM = 15
MSG_DIM = 2 * D
GATE_DIM = D + 2 * NF
CUTOFF = 3.0

_SIGMA = np.arange(D).reshape(NF, IR).T.reshape(-1)
_PI_FULL = np.concatenate([_SIGMA, D + _SIGMA])

_P_TILE = np.tile(np.eye(NF, dtype=np.float32), (1, IR))
_IRP = 384

_R1 = np.zeros((NF, 3 * NF), np.float32)
_R1[np.repeat(np.arange(NF), 3), np.arange(3 * NF)] = 1.0
_R2 = np.zeros((NF, 5 * NF), np.float32)
_R2[np.repeat(np.arange(NF), 5), np.arange(5 * NF)] = 1.0

_DP = 384
_MP = 640
_NC, _NS = 2, 16
_NW = _NC * _NS
_GPW = N_EDGES // _NW
_GCH = 128
_GFULL = _GPW // _GCH
_GTAIL = _GPW - _GFULL * _GCH
_SPS = N_EDGES // _NS
_SCH = 80
_SNCH = _SPS // _SCH
_GRP = 128
_PARTS = (51200, 44800, 38400, 25600)
_NGRP = _MP // _GRP
_WBR = 640
_WCH = 80


def _swish(x):
    return x / (1.0 + jnp.exp(-x))


def _h_body(x_ref, wup_ref, h_ref):
    h_ref[...] = jnp.dot(x_ref[...], wup_ref[...],
                         preferred_element_type=jnp.float32)


def _h_tc(x, wup):
    bn = 2000
    return pl.pallas_call(
        _h_body,
        grid=(N_NODES // bn,),
        in_specs=[
            pl.BlockSpec((bn, D), lambda i: (i, 0)),
            pl.BlockSpec((D, _DP), lambda i: (0, 0)),
        ],
        out_specs=pl.BlockSpec((bn, _DP), lambda i: (i, 0)),
        out_shape=jax.ShapeDtypeStruct((N_NODES, _DP), jnp.float32),
    )(x, wup)


def _sc_body(x_ref, spec_ref, wsc_ref, sc_ref):
    x = x_ref[...]
    spec = spec_ref[...]
    acc = (spec == 0).astype(jnp.float32) * jnp.dot(
        x, wsc_ref[0], preferred_element_type=jnp.float32)
    for s in range(1, 5):
        acc = acc + (spec == s).astype(jnp.float32) * jnp.dot(
            x, wsc_ref[s], preferred_element_type=jnp.float32)
    sc_ref[...] = acc


def _sc_tc(x, spec2d, wsc):
    bn = 2000
    return pl.pallas_call(
        _sc_body,
        grid=(N_NODES // bn,),
        in_specs=[
            pl.BlockSpec((bn, D), lambda i: (i, 0)),
            pl.BlockSpec((bn, 1), lambda i: (i, 0)),
            pl.BlockSpec((5, D, D), lambda i: (0, 0, 0)),
        ],
        out_specs=pl.BlockSpec((bn, D), lambda i: (i, 0)),
        out_shape=jax.ShapeDtypeStruct((N_NODES, D), jnp.float32),
    )(x, spec2d, wsc)


def _make_gather_body(rows_pw):
    nfull = rows_pw // _GCH
    tail = rows_pw - nfull * _GCH
    npair = (nfull - 2) // 2

    def body(h_hbm, snd_hbm, out_hbm, idx_v, rows0, rows1, sem0, sem1):
        wid = lax.axis_index("s") * _NC + lax.axis_index("c")
        base = wid * rows_pw
        pltpu.sync_copy(snd_hbm.at[pl.ds(base, rows_pw)], idx_v)
        bufs, sems = (rows0, rows1), (sem0, sem1)

        def issue(t, buf, sem):
            return pltpu.async_copy(h_hbm.at[idx_v.at[pl.ds(t * _GCH, _GCH)]],
                                    buf, sem)

        def wait(buf, sem):
            pltpu.make_async_copy(h_hbm.at[pl.ds(0, _GCH)], buf, sem).wait()

        def flush(t, buf):
            pltpu.sync_copy(buf, out_hbm.at[pl.ds(base + t * _GCH, _GCH)])

        issue(0, rows0, sem0)
        issue(1, rows1, sem1)

        def pair(p, carry):
            t0 = 2 * p
            wait(rows0, sem0)
            flush(t0, rows0)
            issue(t0 + 2, rows0, sem0)
            wait(rows1, sem1)
            flush(t0 + 1, rows1)
            issue(t0 + 3, rows1, sem1)
            return carry

        lax.fori_loop(0, npair, pair, 0)
        for t in range(2 * npair, nfull):
            b, s = bufs[t % 2], sems[t % 2]
            wait(b, s)
            flush(t, b)
            if t + 2 < nfull:
                issue(t + 2, b, s)
        if tail:
            b, s = bufs[nfull % 2], sems[nfull % 2]
            pltpu.async_copy(
                h_hbm.at[idx_v.at[pl.ds(nfull * _GCH, tail)]],
                b.at[pl.ds(0, tail)], s).wait()
            pltpu.sync_copy(b.at[pl.ds(0, tail)],
                            out_hbm.at[pl.ds(base + nfull * _GCH, tail)])

    return body


def _gather_sc(h, snd, n_e):
    mesh = plsc.VectorSubcoreMesh(core_axis_name="c", subcore_axis_name="s",
                                  num_cores=_NC, num_subcores=_NS)
    f = functools.partial(
        pl.kernel,
        out_type=jax.ShapeDtypeStruct((n_e, _DP), jnp.float32),
        mesh=mesh,
        scratch_types=[
            pltpu.VMEM((n_e // _NW,), jnp.int32),
            pltpu.VMEM((_GCH, _DP), jnp.float32),
            pltpu.VMEM((_GCH, _DP), jnp.float32),
            pltpu.SemaphoreType.DMA,
            pltpu.SemaphoreType.DMA,
        ],
    )(_make_gather_body(n_e // _NW))
    return f(h, snd)


def _edge_body(vec_ref, msgs_ref, wtp_ref, p_ref, w1_ref, w2_ref, w3_ref,
               out_ref):
    vec = vec_ref[...]
    xr, yr, zr = vec[0:1, :], vec[1:2, :], vec[2:3, :]
    l2 = xr * xr + yr * yr + zr * zr
    length = jnp.sqrt(l2)
    inv = 1.0 / length
    ns = lax.broadcasted_iota(jnp.int32, (8, 1), 0).astype(jnp.float32) + 1.0
    t = length * (1.0 / CUTOFF)
    t6 = t * t * t
    t6 = t6 * t6
    env = 1.0 - 28.0 * t6 + 48.0 * t6 * t - 21.0 * t6 * t * t
    env = jnp.where(t < 1.0, env, 0.0)
    radial_t = (jnp.sin(ns * (np.pi / CUTOFF) * length)
                * (np.float32(np.sqrt(2.0 / CUTOFF)) * inv * env))
    x = xr * inv
    y = yr * inv
    z = zr * inv
    x2, y2, z2 = x * x, y * y, z * z
    s3, s5, s7 = np.sqrt(3.0), np.sqrt(5.0), np.sqrt(7.0)
    s15, s105, s42, s70, s358 = (np.sqrt(15.0), np.sqrt(105.0),
                                 np.sqrt(42.0), np.sqrt(70.0),
                                 np.sqrt(35.0 / 8.0))
    sh_t = jnp.concatenate([
        s3 * x, s3 * y, s3 * z,
        s15 * x * y, s15 * y * z, (s5 / 2.0) * (3.0 * z2 - 1.0),
        s15 * x * z, (s15 / 2.0) * (x2 - y2),
        s358 * y * (3.0 * x2 - y2), s105 * x * y * z,
        (s42 / 4.0) * y * (5.0 * z2 - 1.0),
        (s7 / 2.0) * z * (5.0 * z2 - 3.0),
        (s42 / 4.0) * x * (5.0 * z2 - 1.0),
        (s105 / 2.0) * z * (x2 - y2),
        (s70 / 4.0) * x * (x2 - 3.0 * y2),
    ], axis=0)
    b1 = lax.dot_general(sh_t.astype(jnp.bfloat16), wtp_ref[...],
                         (((0,), (0,)), ((), ())),
                         preferred_element_type=jnp.float32
                         ).astype(jnp.bfloat16)
    mix = _swish(lax.dot_general(radial_t, w1_ref[...],
                                 (((0,), (0,)), ((), ())),
                                 preferred_element_type=jnp.float32))
    mix = _swish(jnp.dot(mix, w2_ref[...],
                         preferred_element_type=jnp.float32))
    mix = jnp.dot(mix.astype(jnp.bfloat16), w3_ref[...],
                  preferred_element_type=jnp.float32)
    msgs = msgs_ref[...][:, :D]
    msgs_bf = msgs.astype(jnp.bfloat16)
    p = p_ref[...]
    tp = jnp.dot(msgs_bf[:, 0:NF], p,
                 preferred_element_type=jnp.float32) \
        * b1[:, 0:D].astype(jnp.float32)
    for i in range(1, IR):
        tp = tp + jnp.dot(msgs_bf[:, NF * i:NF * (i + 1)], p,
                          preferred_element_type=jnp.float32) \
            * b1[:, _IRP * i:_IRP * i + D].astype(jnp.float32)
    be = msgs.shape[0]
    out_ref[...] = jnp.concatenate(
        [jnp.concatenate([msgs, tp], axis=1) * mix,
         jnp.zeros((be, _MP - MSG_DIM), jnp.float32)], axis=1)


def _edge_tc(vec_t, msgs, wtpq, ptile, w1, w2, w3p, n_e):
    be = 1280
    return pl.pallas_call(
        _edge_body,
        grid=(n_e // be,),
        in_specs=[
            pl.BlockSpec((3, be), lambda i: (0, i)),
            pl.BlockSpec((be, _DP), lambda i: (i, 0)),
            pl.BlockSpec((SH_DIM, IR * _IRP), lambda i: (0, 0)),
            pl.BlockSpec((NF, D), lambda i: (0, 0)),
            pl.BlockSpec((8, 64), lambda i: (0, 0)),
            pl.BlockSpec((64, 64), lambda i: (0, 0)),
            pl.BlockSpec((64, MSG_DIM), lambda i: (0, 0)),
        ],
        out_specs=pl.BlockSpec((be, _MP), lambda i: (i, 0)),
        out_shape=jax.ShapeDtypeStruct((n_e, _MP), jnp.float32),
    )(vec_t, msgs, wtpq, ptile, w1, w2, w3p)


def _make_scatter_body(eps):
    nfull = eps // _SCH
    npair = (nfull - 2) // 2

    def body(msg_hbm, recv_hbm, z_hbm, agg_hbm,
             idx_v, buf0, buf1, stage, acc_sp, sem0, sem1):
        cid = lax.axis_index("c")
        sid = lax.axis_index("s")
        row0 = sid * _WBR
        nch = jnp.where(sid == _NS - 1, (N_NODES - (_NS - 1) * _WBR) // _WCH,
                        _WBR // _WCH)
        pltpu.sync_copy(recv_hbm.at[sid], idx_v)
        bufs, sems = (buf0, buf1), (sem0, sem1)

        for g_local in range(3):
            col0 = jnp.minimum(cid * 3 + g_local, _NGRP - 1) * _GRP
            pltpu.sync_copy(z_hbm, stage)

            def zero(k, carry):
                pltpu.sync_copy(stage,
                                acc_sp.at[pl.ds(row0 + k * _WCH, _WCH)])
                return carry

            lax.fori_loop(0, nch, zero, 0)
            plsc.subcore_barrier()

            def load(tc, buf, sem):
                return pltpu.async_copy(
                    msg_hbm.at[pl.ds(sid * eps + tc * _SCH, _SCH),
                               pl.ds(col0, _GRP)], buf, sem)

            def wait(buf, sem):
                pltpu.make_async_copy(
                    msg_hbm.at[pl.ds(0, _SCH), pl.ds(0, _GRP)],
                    buf, sem).wait()

            def scat(tc, buf):
                pltpu.sync_copy(buf, acc_sp.at[idx_v.at[tc]], add=True)

            load(0, buf0, sem0)
            load(1, buf1, sem1)

            def pair(p, carry):
                t0 = 2 * p
                wait(buf0, sem0)
                scat(t0, buf0)
                load(t0 + 2, buf0, sem0)
                wait(buf1, sem1)
                scat(t0 + 1, buf1)
                load(t0 + 3, buf1, sem1)
                return carry

            lax.fori_loop(0, npair, pair, 0)
            for t in range(2 * npair, nfull):
                b, s = bufs[t % 2], sems[t % 2]
                wait(b, s)
                scat(t, b)
                if t + 2 < nfull:
                    load(t + 2, b, s)
            plsc.subcore_barrier()

            def wb(k, carry):
                pltpu.sync_copy(acc_sp.at[pl.ds(row0 + k * _WCH, _WCH)],
                                stage)
                pltpu.sync_copy(stage,
                                agg_hbm.at[pl.ds(row0 + k * _WCH, _WCH),
                                           pl.ds(col0, _GRP)])
                return carry

            lax.fori_loop(0, nch, wb, 0)
            plsc.subcore_barrier()

    return body


def _scatter_sc(messages, rcv, zeros_grp, n_e):
    mesh = plsc.VectorSubcoreMesh(core_axis_name="c", subcore_axis_name="s",
                                  num_cores=_NC, num_subcores=_NS)
    eps = n_e // _NS
    f = functools.partial(
        pl.kernel,
        out_type=jax.ShapeDtypeStruct((N_NODES, _MP), jnp.float32),
        mesh=mesh,
        scratch_types=[
            pltpu.VMEM((eps // _SCH, _SCH), jnp.int32),
            pltpu.VMEM((_SCH, _GRP), jnp.float32),
            pltpu.VMEM((_SCH, _GRP), jnp.float32),
            pltpu.VMEM((_WCH, _GRP), jnp.float32),
            pltpu.VMEM_SHARED((N_NODES, _GRP), jnp.float32),
            pltpu.SemaphoreType.DMA,
            pltpu.SemaphoreType.DMA,
        ],
    )(_make_scatter_body(eps))
    return f(messages, rcv, zeros_grp)


def _final_body(*refs):
    (agg_refs, (sc_ref, wd_ref, r1_ref, r2_ref, out_ref)) = (
        refs[:len(_PARTS)], refs[len(_PARTS):])
    agg = agg_refs[0][...][:, :MSG_DIM]
    for r in agg_refs[1:]:
        agg = agg + r[...][:, :MSG_DIM]
    xg = jnp.dot(agg, wd_ref[...], preferred_element_type=jnp.float32)
    s = xg[:, :NF]
    g1 = _swish(xg[:, NF:2 * NF])
    g2 = _swish(xg[:, 2 * NF:3 * NF])
    v1 = xg[:, 3 * NF:6 * NF]
    v2 = xg[:, 6 * NF:]
    s1 = jnp.dot(g1, r1_ref[...], preferred_element_type=jnp.float32)
    s2 = jnp.dot(g2, r2_ref[...], preferred_element_type=jnp.float32)
    out_ref[...] = jnp.concatenate([_swish(s), v1 * s1, v2 * s2],
                                   axis=1) + sc_ref[...]


def _final_tc(aggs, sc, wd, r1, r2):
    bn = 2000
    return pl.pallas_call(
        _final_body,
        grid=(N_NODES // bn,),
        in_specs=[pl.BlockSpec((bn, _MP), lambda i: (i, 0))
                  for _ in aggs] + [
            pl.BlockSpec((bn, D), lambda i: (i, 0)),
            pl.BlockSpec((MSG_DIM, GATE_DIM), lambda i: (0, 0)),
            pl.BlockSpec((NF, 3 * NF), lambda i: (0, 0)),
            pl.BlockSpec((NF, 5 * NF), lambda i: (0, 0)),
        ],
        out_specs=pl.BlockSpec((bn, D), lambda i: (i, 0)),
        out_shape=jax.ShapeDtypeStruct((N_NODES, D), jnp.float32),
    )(*aggs, sc, wd, r1, r2)


def kernel(vectors, node_feats, node_specie, senders, receivers,
           W_sc, W_up, W_tp, W1, W2, W3, W_down):
    wup_p = jnp.pad(W_up[:, _SIGMA], ((0, 0), (0, _DP - D)))
    wtp_r = jnp.transpose(W_tp, (1, 0, 2)).reshape(SH_DIM, IR * IR)
    src = np.concatenate([np.clip(np.arange(_IRP) // NF, 0, IR - 1) + IR * i
                          for i in range(IR)])
    msk = np.concatenate([(np.arange(_IRP) < D).astype(np.float32)] * IR)
    wtpq = (wtp_r[:, src] * msk[None, :]).astype(jnp.bfloat16)
    ptile = jnp.asarray(_P_TILE).astype(jnp.bfloat16)
    w3_p = W3[:, _PI_FULL].astype(jnp.bfloat16)
    wd_p = W_down[_PI_FULL, :] * np.float32(0.25)
    r1 = jnp.asarray(_R1)
    r2 = jnp.asarray(_R2)
    spec2d = node_specie.astype(jnp.int32).reshape(N_NODES, 1)
    snd = senders.astype(jnp.int32)
    rcv = receivers.astype(jnp.int32)
    zeros_grp = jnp.zeros((_WCH, _GRP), jnp.float32)
    vec_t = jnp.transpose(vectors)

    h = _h_tc(node_feats, wup_p)
    sc = _sc_tc(node_feats, spec2d, W_sc)
    aggs = []
    off = 0
    for ne in _PARTS:
        sl = slice(off, off + ne)
        msgs = _gather_sc(h, snd[sl], ne)
        m = _edge_tc(vec_t[:, sl], msgs, wtpq, ptile, W1, W2, w3_p, ne)
        aggs.append(_scatter_sc(
            m, rcv[sl].reshape(_NS, ne // _NS // _SCH, _SCH),
            zeros_grp, ne))
        off += ne
    return _final_tc(aggs, sc, wd_p, r1, r2)
